# speedup vs baseline: 1.0520x; 1.0520x over previous
import functools

import jax
import jax.numpy as jnp
from jax.experimental import pallas as pl
from jax.experimental.pallas import tpu as pltpu

N_NODES = 10000
BN = 1000


def _mm_body(x_ref, w_ref, b_ref, o_ref, *, act):
    acc = jnp.dot(x_ref[...], w_ref[...], preferred_element_type=jnp.float32)
    acc = acc + b_ref[...][None, :]
    if act == "relu":
        acc = jnp.maximum(acc, 0.0)
    elif act == "sigmoid":
        acc = jax.nn.sigmoid(acc)
    o_ref[...] = acc


def _mm(x, w, b, act="none"):
    n, k = x.shape
    m = w.shape[1]
    grid = n // BN
    return pl.pallas_call(
        functools.partial(_mm_body, act=act),
        grid=(grid,),
        in_specs=[
            pl.BlockSpec((BN, k), lambda i: (i, 0)),
            pl.BlockSpec((k, m), lambda i: (0, 0)),
            pl.BlockSpec((m,), lambda i: (0,)),
        ],
        out_specs=pl.BlockSpec((BN, m), lambda i: (i, 0)),
        out_shape=jax.ShapeDtypeStruct((n, m), jnp.float32),
    )(x, w, b)


def _gat_edges(h, alpha_s, alpha_d, src, dst, heads, out_ch):
    n = h.shape[0]
    e = jax.nn.leaky_relu(alpha_s[src] + alpha_d[dst], 0.2)
    e_exp = jnp.exp(e)
    denom = jax.ops.segment_sum(e_exp, dst, num_segments=n)
    alpha = e_exp / denom[dst]
    hh = h.reshape(n, heads, out_ch)
    out = jax.ops.segment_sum(hh[src] * alpha[:, :, None], dst, num_segments=n)
    return out.reshape(n, heads * out_ch)


def _fold_attn(w, a_src, a_dst, heads, out_ch):
    wr = w.reshape(w.shape[0], heads, out_ch)
    ws = (wr * a_src[None]).sum(-1)
    wd = (wr * a_dst[None]).sum(-1)
    return jnp.concatenate([w, ws, wd], axis=1)


def kernel(x, params, edge_index):
    p = params
    n = x.shape[0]
    loop = jnp.arange(n, dtype=edge_index.dtype)
    src = jnp.concatenate([edge_index[0], loop])
    dst = jnp.concatenate([edge_index[1], loop])

    w1 = _fold_attn(p["c1_w"], p["c1_as"], p["c1_ad"], 8, 32)
    m1 = _mm(x, w1, jnp.zeros((w1.shape[1],), jnp.float32))
    h1, as1, ad1 = m1[:, :256], m1[:, 256:264], m1[:, 264:272]
    agg1 = _gat_edges(h1, as1, ad1, src, dst, 8, 32)
    x1 = jnp.maximum(agg1 + p["c1_b"], 0.0)

    w2 = _fold_attn(p["c2_w"], p["c2_as"], p["c2_ad"], 8, 32)
    m2 = _mm(x1, w2, jnp.zeros((w2.shape[1],), jnp.float32))
    h2, as2, ad2 = m2[:, :256], m2[:, 256:264], m2[:, 264:272]
    agg2 = _gat_edges(h2, as2, ad2, src, dst, 8, 32)
    x2 = jnp.maximum(agg2 + p["c2_b"], 0.0)

    w3 = _fold_attn(p["c3_w"], p["c3_as"], p["c3_ad"], 12, 64)
    m3 = _mm(x2, w3, jnp.zeros((w3.shape[1],), jnp.float32))
    h3, as3, ad3 = m3[:, :768], m3[:, 768:780], m3[:, 780:792]
    agg3 = _gat_edges(h3, as3, ad3, src, dst, 12, 64)
    x3 = jnp.maximum(agg3 + p["c3_b"], 0.0)

    gg1 = _mm(x, p["ln1_w"], p["ln1_b"], act="relu")
    gg2 = _mm(gg1, p["ln2_w"], p["ln2_b"], act="relu")
    xa1 = _mm(x1, p["a1_w"], p["a1_b"], act="relu")
    xa2 = _mm(x2, p["a3_w"], p["a3_b"], act="relu")

    xf = jnp.concatenate([gg2, x3, xa1, xa2], axis=1)
    xf = _mm(xf, p["f1_w"], p["f1_b"], act="relu")
    xf = _mm(xf, p["f2_w"], p["f2_b"], act="relu")
    out = _mm(xf, p["f3_w"], p["f3_b"], act="sigmoid")
    return out


# speedup vs baseline: 11.8136x; 11.2296x over previous
import functools

import jax
import jax.numpy as jnp
from jax import lax
from jax.experimental import pallas as pl
from jax.experimental.pallas import tpu as pltpu
from jax.experimental.pallas import tpu_sc as plsc

N_NODES = 10000
N_PAD = 10240
E_REAL = 330000
KB = 128
NB = 168
PT = NB * KB
E_PAD = 16 * PT
ROWS_T = N_PAD // 16
BN = 1000


def _mm_body(x_ref, w_ref, b_ref, o_ref, *, act):
    acc = jnp.dot(x_ref[...], w_ref[...], preferred_element_type=jnp.float32)
    acc = acc + b_ref[...][None, :]
    if act == "relu":
        acc = jnp.maximum(acc, 0.0)
    o_ref[...] = acc


def _mm(x, w, b, act="none"):
    n, k = x.shape
    m = w.shape[1]
    return pl.pallas_call(
        functools.partial(_mm_body, act=act),
        grid=(n // BN,),
        in_specs=[
            pl.BlockSpec((BN, k), lambda i: (i, 0)),
            pl.BlockSpec((k, m), lambda i: (0, 0)),
            pl.BlockSpec((m,), lambda i: (0,)),
        ],
        out_specs=pl.BlockSpec((BN, m), lambda i: (i, 0)),
        out_shape=jax.ShapeDtypeStruct((n, m), jnp.float32),
    )(x, w, b)


def _tc1_body(x_ref, w1_ref, ln1w_ref, ln1b_ref, ln2w_ref, ln2b_ref,
              m1_ref, gg2_ref):
    x = x_ref[...]
    m1_ref[...] = jnp.dot(x, w1_ref[...], preferred_element_type=jnp.float32)
    gg1 = jnp.maximum(jnp.dot(x, ln1w_ref[...],
                              preferred_element_type=jnp.float32)
                      + ln1b_ref[...][None, :], 0.0)
    gg2_ref[...] = jnp.maximum(jnp.dot(gg1, ln2w_ref[...],
                                       preferred_element_type=jnp.float32)
                               + ln2b_ref[...][None, :], 0.0)


def _tc_mid_body(acc_ref, den_ref, exp_ref, bias_ref, w_ref, wa_ref, ba_ref,
                 m_ref, xa_ref, *, nc):
    acc = jnp.concatenate([acc_ref[i] for i in range(nc)], axis=-1)
    dd = jnp.dot(den_ref[...], exp_ref[...],
                 preferred_element_type=jnp.float32)
    xl = jnp.maximum(acc / dd + bias_ref[...][None, :], 0.0)
    m_ref[...] = jnp.dot(xl, w_ref[...], preferred_element_type=jnp.float32)
    xa_ref[...] = jnp.maximum(jnp.dot(xl, wa_ref[...],
                                      preferred_element_type=jnp.float32)
                              + ba_ref[...][None, :], 0.0)


def _tc_head_body(acc_ref, den_ref, exp_ref, bias_ref, gg2_ref, xa1_ref,
                  xa2_ref, f1x_ref, f1g_ref, f1a1_ref, f1a2_ref, f1b_ref,
                  f2w_ref, f2b_ref, f3w_ref, f3b_ref, o_ref, *, nc):
    acc = jnp.concatenate([acc_ref[i] for i in range(nc)], axis=-1)
    dd = jnp.dot(den_ref[...], exp_ref[...],
                 preferred_element_type=jnp.float32)
    x3 = jnp.maximum(acc / dd + bias_ref[...][None, :], 0.0)
    xf = (jnp.dot(x3, f1x_ref[...], preferred_element_type=jnp.float32)
          + jnp.dot(gg2_ref[...], f1g_ref[...],
                    preferred_element_type=jnp.float32)
          + jnp.dot(xa1_ref[...], f1a1_ref[...],
                    preferred_element_type=jnp.float32)
          + jnp.dot(xa2_ref[...], f1a2_ref[...],
                    preferred_element_type=jnp.float32)
          + f1b_ref[...][None, :])
    xf = jnp.maximum(xf, 0.0)
    xf = jnp.maximum(jnp.dot(xf, f2w_ref[...],
                             preferred_element_type=jnp.float32)
                     + f2b_ref[...][None, :], 0.0)
    xf = jnp.dot(xf, f3w_ref[...], preferred_element_type=jnp.float32) \
        + f3b_ref[...][None, :]
    o_ref[...] = jax.nn.sigmoid(xf)


def _rep(k, m):
    return pl.BlockSpec((k, m), lambda i: (0, 0))


def _vec(m):
    return pl.BlockSpec((m,), lambda i: (0,))


def _rows(m):
    return pl.BlockSpec((BN, m), lambda i: (i, 0))


CW = 64
NV = CW // 16


@functools.lru_cache(maxsize=None)
def _make_gat_sc(heads, ch):
    shift = 5 if ch == 32 else 6
    nc = heads * ch // CW
    n_rounds = nc // 2
    mesh = plsc.VectorSubcoreMesh(core_axis_name="c", subcore_axis_name="s",
                                  num_cores=2, num_subcores=16)

    def body(h4, atab, dtab, srcr, dstr, acc_hbm, den_hbm,
             src2d, dst2d, asb, adb, wb, hbuf, ridx, zb, zbd,
             acc, den, sem1, sem2):
        g = lax.axis_index("c")
        sid = lax.axis_index("s")
        rowblk = sid * NB
        pltpu.sync_copy(srcr.at[pl.ds(rowblk, NB)], src2d)
        pltpu.sync_copy(dstr.at[pl.ds(rowblk, NB)], dst2d)

        zv = jnp.zeros((16,), jnp.float32)

        def z1(i, carry):
            for v in range(NV):
                zb[i, pl.ds(v * 16, 16)] = zv
            return carry

        lax.fori_loop(0, 128, z1, 0)

        def z2(i, carry):
            zbd[i] = zv
            return carry

        lax.fori_loop(0, ROWS_T, z2, 0)

        nrow0 = sid * ROWS_T
        pltpu.sync_copy(zbd, den.at[pl.ds(nrow0, ROWS_T)])

        iota = lax.iota(jnp.int32, 16)

        for r in range(n_rounds):
            c = g * n_rounds + r
            for s5 in range(5):
                pltpu.sync_copy(zb, acc.at[pl.ds(nrow0 + s5 * 128, 128)])
            plsc.subcore_barrier()

            hgs = [jnp.full((16,), lax.shift_right_logical(c * CW + v * 16,
                                                           shift), jnp.int32)
                   for v in range(NV)]

            def blk(b, carry):
                ebase = sid * PT + b * KB
                pltpu.async_copy(atab.at[src2d.at[b]], asb, sem1).wait()
                pltpu.async_copy(dtab.at[dst2d.at[b]], adb, sem1).wait()

                def rix(i, cc):
                    sv = src2d[b, pl.ds(i * 16, 16)]
                    ridx[pl.ds(i * 16, 16)] = sv * nc + c
                    return cc

                lax.fori_loop(0, KB // 16, rix, 0)
                hdma = pltpu.async_copy(h4.at[ridx], hbuf, sem2)

                def edge(e, cc):
                    ev = asb[e] + adb[e]
                    ev = jnp.where(ev >= 0.0, ev, 0.2 * ev)
                    wv = jnp.exp(ev)
                    scale = jnp.where(ebase + e < E_REAL, 1.0, 0.0)
                    wb[e] = wv * scale
                    return cc

                lax.fori_loop(0, KB, edge, 0)
                if r == 0:
                    pltpu.sync_copy(wb, den.at[dst2d.at[b]], add=True)
                hdma.wait()

                def edge2(e, cc):
                    wv = wb[e]
                    for v in range(NV):
                        al = wv.at[hgs[v]].get(mode="promise_in_bounds")
                        hv = hbuf[e, pl.ds(v * 16, 16)]
                        hbuf[e, pl.ds(v * 16, 16)] = hv * al
                    return cc

                lax.fori_loop(0, KB, edge2, 0)
                pltpu.sync_copy(hbuf, acc.at[dst2d.at[b]], add=True)
                return carry

            lax.fori_loop(0, NB, blk, 0)
            plsc.subcore_barrier()
            pltpu.sync_copy(acc.at[pl.ds(nrow0, ROWS_T)],
                            acc_hbm.at[c, pl.ds(nrow0, ROWS_T)])
            if r == 0:
                @pl.when(g == 0)
                def _dump_den():
                    pltpu.sync_copy(den.at[pl.ds(nrow0, ROWS_T)],
                                    den_hbm.at[pl.ds(nrow0, ROWS_T)])

    return pl.kernel(
        body,
        out_type=(
            jax.ShapeDtypeStruct((nc, N_PAD, CW), jnp.float32),
            jax.ShapeDtypeStruct((N_PAD, 16), jnp.float32),
        ),
        mesh=mesh,
        scratch_types=[
            pltpu.VMEM((NB, KB), jnp.int32),
            pltpu.VMEM((NB, KB), jnp.int32),
            pltpu.VMEM((KB, 16), jnp.float32),
            pltpu.VMEM((KB, 16), jnp.float32),
            pltpu.VMEM((KB, 16), jnp.float32),
            pltpu.VMEM((KB, CW), jnp.float32),
            pltpu.VMEM((KB,), jnp.int32),
            pltpu.VMEM((128, CW), jnp.float32),
            pltpu.VMEM((ROWS_T, 16), jnp.float32),
            pltpu.VMEM_SHARED((N_PAD, CW), jnp.float32),
            pltpu.VMEM_SHARED((N_PAD, 16), jnp.float32),
            pltpu.SemaphoreType.DMA,
            pltpu.SemaphoreType.DMA,
        ],
        compiler_params=pltpu.CompilerParams(use_tc_tiling_on_sc=False),
    )


def _fold_attn(w, a_src, a_dst, heads, out_ch):
    wr = w.reshape(w.shape[0], heads, out_ch)
    ws = (wr * a_src[None]).sum(-1)
    wd = (wr * a_dst[None]).sum(-1)
    return jnp.concatenate([w, ws, wd], axis=1)


def _expand_mat(heads, ch):
    e = jnp.zeros((16, heads * ch), jnp.float32)
    r = jnp.arange(heads * ch) // ch
    e = e.at[r, jnp.arange(heads * ch)].set(1.0)
    return e


def _split_m(m, heads, ch):
    hc = heads * ch
    nc = hc // CW
    h4 = m[:, :hc].reshape(N_NODES * nc, CW)
    atab = jnp.pad(m[:, hc:hc + heads], ((0, 0), (0, 16 - heads)))
    dtab = jnp.pad(m[:, hc + heads:hc + 2 * heads], ((0, 0), (0, 16 - heads)))
    return h4, atab, dtab


def kernel(x, params, edge_index):
    p = params
    n = N_NODES
    loop = jnp.arange(n, dtype=edge_index.dtype)
    padz = jnp.zeros((E_PAD - E_REAL,), edge_index.dtype)
    srcr = jnp.concatenate([edge_index[0], loop, padz]).reshape(16 * NB, KB)
    dstr = jnp.concatenate([edge_index[1], loop, padz]).reshape(16 * NB, KB)

    w1 = _fold_attn(p["c1_w"], p["c1_as"], p["c1_ad"], 8, 32)
    w2 = _fold_attn(p["c2_w"], p["c2_as"], p["c2_ad"], 8, 32)
    w3 = _fold_attn(p["c3_w"], p["c3_as"], p["c3_ad"], 12, 64)
    exp8 = _expand_mat(8, 32)
    exp12 = _expand_mat(12, 64)

    m1, gg2 = pl.pallas_call(
        _tc1_body,
        grid=(n // BN,),
        in_specs=[_rows(128), _rep(128, 272), _rep(128, 32), _vec(32),
                  _rep(32, 32), _vec(32)],
        out_specs=[_rows(272), _rows(32)],
        out_shape=[jax.ShapeDtypeStruct((n, 272), jnp.float32),
                   jax.ShapeDtypeStruct((n, 32), jnp.float32)],
    )(x, w1, p["ln1_w"], p["ln1_b"], p["ln2_w"], p["ln2_b"])

    gat1_sc = _make_gat_sc(8, 32)
    gat3_sc = _make_gat_sc(12, 64)

    h4, atab, dtab = _split_m(m1, 8, 32)
    acc1, den1 = gat1_sc(h4, atab, dtab, srcr, dstr)
    acc1, den1 = acc1[:, :n], den1[:n]

    m2, xa1 = pl.pallas_call(
        functools.partial(_tc_mid_body, nc=4),
        grid=(n // BN,),
        in_specs=[pl.BlockSpec((4, BN, CW), lambda i: (0, i, 0)),
                  _rows(16), _rep(16, 256), _vec(256),
                  _rep(256, 272), _rep(256, 80), _vec(80)],
        out_specs=[_rows(272), _rows(80)],
        out_shape=[jax.ShapeDtypeStruct((n, 272), jnp.float32),
                   jax.ShapeDtypeStruct((n, 80), jnp.float32)],
    )(acc1, den1, exp8, p["c1_b"], w2, p["a1_w"], p["a1_b"])

    h4, atab, dtab = _split_m(m2, 8, 32)
    acc2, den2 = gat1_sc(h4, atab, dtab, srcr, dstr)
    acc2, den2 = acc2[:, :n], den2[:n]

    m3, xa2 = pl.pallas_call(
        functools.partial(_tc_mid_body, nc=4),
        grid=(n // BN,),
        in_specs=[pl.BlockSpec((4, BN, CW), lambda i: (0, i, 0)),
                  _rows(16), _rep(16, 256), _vec(256),
                  _rep(256, 792), _rep(256, 200), _vec(200)],
        out_specs=[_rows(792), _rows(200)],
        out_shape=[jax.ShapeDtypeStruct((n, 792), jnp.float32),
                   jax.ShapeDtypeStruct((n, 200), jnp.float32)],
    )(acc2, den2, exp8, p["c2_b"], w3, p["a3_w"], p["a3_b"])

    h4, atab, dtab = _split_m(m3, 12, 64)
    acc3, den3 = gat3_sc(h4, atab, dtab, srcr, dstr)
    acc3, den3 = acc3[:, :n], den3[:n]

    f1 = p["f1_w"]
    out = pl.pallas_call(
        functools.partial(_tc_head_body, nc=12),
        grid=(n // BN,),
        in_specs=[pl.BlockSpec((12, BN, CW), lambda i: (0, i, 0)),
                  _rows(16), _rep(16, 768), _vec(768),
                  _rows(32), _rows(80), _rows(200),
                  _rep(768, 200), _rep(32, 200), _rep(80, 200),
                  _rep(200, 200), _vec(200),
                  _rep(200, 64), _vec(64), _rep(64, 1), _vec(1)],
        out_specs=_rows(1),
        out_shape=jax.ShapeDtypeStruct((n, 1), jnp.float32),
    )(acc3, den3, exp12, p["c3_b"], gg2, xa1, xa2,
      f1[32:800], f1[:32], f1[800:880], f1[880:1080], p["f1_b"],
      p["f2_w"], p["f2_b"], p["f3_w"], p["f3_b"])
    return out


# speedup vs baseline: 20.2162x; 1.7113x over previous
import functools

import jax
import jax.numpy as jnp
from jax import lax
from jax.experimental import pallas as pl
from jax.experimental.pallas import tpu as pltpu
from jax.experimental.pallas import tpu_sc as plsc

N_NODES = 10000
N_PAD = 10240
E_REAL = 330000
KB = 128
NB = 168
PT = NB * KB
E_PAD = 16 * PT
ROWS_T = N_PAD // 16
BN = 1000


def _mm_body(x_ref, w_ref, b_ref, o_ref, *, act):
    acc = jnp.dot(x_ref[...], w_ref[...], preferred_element_type=jnp.float32)
    acc = acc + b_ref[...][None, :]
    if act == "relu":
        acc = jnp.maximum(acc, 0.0)
    o_ref[...] = acc


def _mm(x, w, b, act="none"):
    n, k = x.shape
    m = w.shape[1]
    return pl.pallas_call(
        functools.partial(_mm_body, act=act),
        grid=(n // BN,),
        in_specs=[
            pl.BlockSpec((BN, k), lambda i: (i, 0)),
            pl.BlockSpec((k, m), lambda i: (0, 0)),
            pl.BlockSpec((m,), lambda i: (0,)),
        ],
        out_specs=pl.BlockSpec((BN, m), lambda i: (i, 0)),
        out_shape=jax.ShapeDtypeStruct((n, m), jnp.float32),
    )(x, w, b)


def _tc1_body(x_ref, w1_ref, ln1w_ref, ln1b_ref, ln2w_ref, ln2b_ref,
              m1_ref, gg2_ref):
    x = x_ref[...]
    m1_ref[...] = jnp.dot(x, w1_ref[...], preferred_element_type=jnp.float32)
    gg1 = jnp.maximum(jnp.dot(x, ln1w_ref[...],
                              preferred_element_type=jnp.float32)
                      + ln1b_ref[...][None, :], 0.0)
    gg2_ref[...] = jnp.maximum(jnp.dot(gg1, ln2w_ref[...],
                                       preferred_element_type=jnp.float32)
                               + ln2b_ref[...][None, :], 0.0)


def _tc_mid_body(acc_ref, den_ref, exp_ref, bias_ref, w_ref, wa_ref, ba_ref,
                 m_ref, xa_ref, *, nc):
    acc = jnp.concatenate([acc_ref[i] for i in range(nc)], axis=-1)
    dd = jnp.dot(den_ref[...], exp_ref[...],
                 preferred_element_type=jnp.float32)
    xl = jnp.maximum(acc / dd + bias_ref[...][None, :], 0.0)
    m_ref[...] = jnp.dot(xl, w_ref[...], preferred_element_type=jnp.float32)
    xa_ref[...] = jnp.maximum(jnp.dot(xl, wa_ref[...],
                                      preferred_element_type=jnp.float32)
                              + ba_ref[...][None, :], 0.0)


def _tc_head_body(acc_ref, den_ref, exp_ref, bias_ref, gg2_ref, xa1_ref,
                  xa2_ref, f1x_ref, f1g_ref, f1a1_ref, f1a2_ref, f1b_ref,
                  f2w_ref, f2b_ref, f3w_ref, f3b_ref, o_ref, *, nc):
    acc = jnp.concatenate([acc_ref[i] for i in range(nc)], axis=-1)
    dd = jnp.dot(den_ref[...], exp_ref[...],
                 preferred_element_type=jnp.float32)
    x3 = jnp.maximum(acc / dd + bias_ref[...][None, :], 0.0)
    xf = (jnp.dot(x3, f1x_ref[...], preferred_element_type=jnp.float32)
          + jnp.dot(gg2_ref[...], f1g_ref[...],
                    preferred_element_type=jnp.float32)
          + jnp.dot(xa1_ref[...], f1a1_ref[...],
                    preferred_element_type=jnp.float32)
          + jnp.dot(xa2_ref[...], f1a2_ref[...],
                    preferred_element_type=jnp.float32)
          + f1b_ref[...][None, :])
    xf = jnp.maximum(xf, 0.0)
    xf = jnp.maximum(jnp.dot(xf, f2w_ref[...],
                             preferred_element_type=jnp.float32)
                     + f2b_ref[...][None, :], 0.0)
    xf = jnp.dot(xf, f3w_ref[...], preferred_element_type=jnp.float32) \
        + f3b_ref[...][None, :]
    o_ref[...] = jax.nn.sigmoid(xf)


def _rep(k, m):
    return pl.BlockSpec((k, m), lambda i: (0, 0))


def _vec(m):
    return pl.BlockSpec((m,), lambda i: (0,))


def _rows(m):
    return pl.BlockSpec((BN, m), lambda i: (i, 0))


CW = 64
NV = CW // 16


@functools.lru_cache(maxsize=None)
def _make_gat_sc(heads, ch):
    shift = 5 if ch == 32 else 6
    nc = heads * ch // CW
    n_rounds = nc // 2
    n_al = CW // ch if ch < CW else 1
    mesh = plsc.VectorSubcoreMesh(core_axis_name="c", subcore_axis_name="s",
                                  num_cores=2, num_subcores=16)

    def body(h4, atab, dtab, srcr, dstr, acc_hbm, den_hbm, w_hbm,
             srcb, dstb, asb, adb, wb, hbuf, obuf, ridx, zb, zbd,
             acc, den, sems):
        g = lax.axis_index("c")
        sid = lax.axis_index("s")
        rowblk = sid * NB

        zv = jnp.zeros((16,), jnp.float32)

        def z1(i, carry):
            for v in range(NV):
                zb[i, pl.ds(v * 16, 16)] = zv
            return carry

        lax.fori_loop(0, 128, z1, 0)

        def z2(i, carry):
            zbd[i] = zv
            return carry

        lax.fori_loop(0, ROWS_T, z2, 0)

        nrow0 = sid * ROWS_T
        pltpu.sync_copy(zbd, den.at[pl.ds(nrow0, ROWS_T)])

        ebase0 = sid * PT
        sem_a, sem_d, sem_h, sem_ws, sem_ww, sem_ha, sem_e = sems

        def drain(kind, sl):
            if kind == "a":
                pltpu.make_async_copy(atab.at[pl.ds(0, KB)], asb.at[sl],
                                      sem_a.at[sl]).wait()
            elif kind == "d":
                pltpu.make_async_copy(dtab.at[pl.ds(0, KB)], adb.at[sl],
                                      sem_d.at[sl]).wait()
            elif kind == "h":
                pltpu.make_async_copy(h4.at[pl.ds(0, KB)], hbuf.at[sl],
                                      sem_h.at[sl]).wait()
            elif kind == "ws":
                pltpu.make_async_copy(wb.at[sl], den.at[pl.ds(0, KB)],
                                      sem_ws.at[sl]).wait()
            elif kind == "ww":
                pltpu.make_async_copy(wb.at[sl], w_hbm.at[pl.ds(0, KB)],
                                      sem_ww.at[sl]).wait()
            elif kind == "ha":
                pltpu.make_async_copy(obuf.at[sl], acc.at[pl.ds(0, KB)],
                                      sem_ha.at[sl]).wait()
            elif kind == "e":
                pltpu.make_async_copy(srcr.at[0], srcb.at[sl],
                                      sem_e.at[sl]).wait()
                pltpu.make_async_copy(srcr.at[0], srcb.at[sl],
                                      sem_e.at[sl]).wait()

        def pf_edges(b):
            el = jnp.bitwise_and(b, 7)
            pltpu.async_copy(srcr.at[rowblk + b], srcb.at[el], sem_e.at[el])
            pltpu.async_copy(dstr.at[rowblk + b], dstb.at[el], sem_e.at[el])

        for r in range(n_rounds):
            c = g * n_rounds + r
            for s5 in range(5):
                pltpu.sync_copy(zb, acc.at[pl.ds(nrow0 + s5 * 128, 128)])
            plsc.subcore_barrier()

            hgs = [jnp.full((16,), lax.shift_right_logical(
                c * CW + a * ch, shift), jnp.int32) for a in range(n_al)]

            def pf_gather(b, sl):
                el = jnp.bitwise_and(b, 7)
                drain("e", el)
                if r == 0:
                    pltpu.async_copy(atab.at[srcb.at[el]], asb.at[sl],
                                     sem_a.at[sl])
                    pltpu.async_copy(dtab.at[dstb.at[el]], adb.at[sl],
                                     sem_d.at[sl])
                else:
                    pltpu.async_copy(
                        w_hbm.at[pl.ds(ebase0 + b * KB, KB)], wb.at[sl],
                        sem_a.at[sl])

                def rix(i, cc):
                    sv = srcb[el, pl.ds(i * 16, 16)]
                    ridx[sl, pl.ds(i * 16, 16)] = sv * nc + c
                    return cc

                lax.fori_loop(0, KB // 16, rix, 0)
                pltpu.async_copy(h4.at[ridx.at[sl]], hbuf.at[sl],
                                 sem_h.at[sl])

            pf_edges(0)
            pf_edges(1)
            pf_edges(2)
            pf_gather(0, 0)
            pf_gather(1, 1)

            def section(j, b, sl):
                el = jnp.bitwise_and(b, 7)
                not_first = j > 0
                drain("a", sl)
                if r == 0:
                    drain("d", sl)
                drain("h", sl)

                @pl.when(not_first)
                def _drains():
                    if r == 0:
                        drain("ws", sl)
                        drain("ww", sl)
                    drain("ha", sl)

                if r == 0:
                    ebase = ebase0 + b * KB

                    def edge(e, cc):
                        ev = asb[sl, e] + adb[sl, e]
                        ev = jnp.where(ev >= 0.0, ev, 0.2 * ev)
                        scale = jnp.where(ebase + e < E_REAL, 1.0, 0.0)
                        wv = jnp.exp(ev) * scale
                        wb[sl, e] = wv
                        for v in range(NV):
                            al = wv.at[hgs[(v * 16) // ch]].get(
                                mode="promise_in_bounds")
                            obuf[sl, e, pl.ds(v * 16, 16)] = (
                                hbuf[sl, e, pl.ds(v * 16, 16)] * al)
                        return cc

                    lax.fori_loop(0, KB, edge, 0)
                    pltpu.async_copy(wb.at[sl], den.at[dstb.at[el]],
                                     sem_ws.at[sl], add=True)
                    pltpu.async_copy(wb.at[sl],
                                     w_hbm.at[pl.ds(ebase0 + b * KB, KB)],
                                     sem_ww.at[sl])
                else:
                    def edge(e, cc):
                        wv = wb[sl, e]
                        for v in range(NV):
                            al = wv.at[hgs[(v * 16) // ch]].get(
                                mode="promise_in_bounds")
                            obuf[sl, e, pl.ds(v * 16, 16)] = (
                                hbuf[sl, e, pl.ds(v * 16, 16)] * al)
                        return cc

                    lax.fori_loop(0, KB, edge, 0)

                pltpu.async_copy(obuf.at[sl], acc.at[dstb.at[el]],
                                 sem_ha.at[sl], add=True)

                @pl.when(b + 3 < NB)
                def _pfe():
                    pf_edges(b + 3)

                @pl.when(b + 2 < NB)
                def _pf():
                    pf_gather(b + 2, sl)

            def blk2(j, carry):
                section(j, 2 * j, 0)
                section(j, 2 * j + 1, 1)
                return carry

            lax.fori_loop(0, NB // 2, blk2, 0)

            for sl in range(2):
                if r == 0:
                    drain("ws", sl)
                    drain("ww", sl)
                drain("ha", sl)
            plsc.subcore_barrier()
            pltpu.sync_copy(acc.at[pl.ds(nrow0, ROWS_T)],
                            acc_hbm.at[c, pl.ds(nrow0, ROWS_T)])
            if r == 0:
                @pl.when(g == 0)
                def _dump_den():
                    pltpu.sync_copy(den.at[pl.ds(nrow0, ROWS_T)],
                                    den_hbm.at[pl.ds(nrow0, ROWS_T)])

    return pl.kernel(
        body,
        out_type=(
            jax.ShapeDtypeStruct((nc, N_PAD, CW), jnp.float32),
            jax.ShapeDtypeStruct((N_PAD, 16), jnp.float32),
            jax.ShapeDtypeStruct((E_PAD, 16), jnp.float32),
        ),
        mesh=mesh,
        scratch_types=[
            pltpu.VMEM((8, KB), jnp.int32),
            pltpu.VMEM((8, KB), jnp.int32),
            pltpu.VMEM((2, KB, 16), jnp.float32),
            pltpu.VMEM((2, KB, 16), jnp.float32),
            pltpu.VMEM((2, KB, 16), jnp.float32),
            pltpu.VMEM((2, KB, CW), jnp.float32),
            pltpu.VMEM((2, KB, CW), jnp.float32),
            pltpu.VMEM((2, KB), jnp.int32),
            pltpu.VMEM((128, CW), jnp.float32),
            pltpu.VMEM((ROWS_T, 16), jnp.float32),
            pltpu.VMEM_SHARED((N_PAD, CW), jnp.float32),
            pltpu.VMEM_SHARED((N_PAD, 16), jnp.float32),
            [pltpu.SemaphoreType.DMA((2,))] * 6
            + [pltpu.SemaphoreType.DMA((8,))],
        ],
        compiler_params=pltpu.CompilerParams(use_tc_tiling_on_sc=False),
    )


def _fold_attn(w, a_src, a_dst, heads, out_ch):
    wr = w.reshape(w.shape[0], heads, out_ch)
    ws = (wr * a_src[None]).sum(-1)
    wd = (wr * a_dst[None]).sum(-1)
    return jnp.concatenate([w, ws, wd], axis=1)


def _expand_mat(heads, ch):
    e = jnp.zeros((16, heads * ch), jnp.float32)
    r = jnp.arange(heads * ch) // ch
    e = e.at[r, jnp.arange(heads * ch)].set(1.0)
    return e


def _split_m(m, heads, ch):
    hc = heads * ch
    nc = hc // CW
    h4 = m[:, :hc].reshape(N_NODES * nc, CW)
    atab = jnp.pad(m[:, hc:hc + heads], ((0, 0), (0, 16 - heads)))
    dtab = jnp.pad(m[:, hc + heads:hc + 2 * heads], ((0, 0), (0, 16 - heads)))
    return h4, atab, dtab


def kernel(x, params, edge_index):
    p = params
    n = N_NODES
    loop = jnp.arange(n, dtype=edge_index.dtype)
    padz = jnp.zeros((E_PAD - E_REAL,), edge_index.dtype)
    srcr = jnp.concatenate([edge_index[0], loop, padz]).reshape(16 * NB, KB)
    dstr = jnp.concatenate([edge_index[1], loop, padz]).reshape(16 * NB, KB)

    w1 = _fold_attn(p["c1_w"], p["c1_as"], p["c1_ad"], 8, 32)
    w2 = _fold_attn(p["c2_w"], p["c2_as"], p["c2_ad"], 8, 32)
    w3 = _fold_attn(p["c3_w"], p["c3_as"], p["c3_ad"], 12, 64)
    exp8 = _expand_mat(8, 32)
    exp12 = _expand_mat(12, 64)

    m1, gg2 = pl.pallas_call(
        _tc1_body,
        grid=(n // BN,),
        in_specs=[_rows(128), _rep(128, 272), _rep(128, 32), _vec(32),
                  _rep(32, 32), _vec(32)],
        out_specs=[_rows(272), _rows(32)],
        out_shape=[jax.ShapeDtypeStruct((n, 272), jnp.float32),
                   jax.ShapeDtypeStruct((n, 32), jnp.float32)],
    )(x, w1, p["ln1_w"], p["ln1_b"], p["ln2_w"], p["ln2_b"])

    gat1_sc = _make_gat_sc(8, 32)
    gat3_sc = _make_gat_sc(12, 64)

    h4, atab, dtab = _split_m(m1, 8, 32)
    acc1, den1, _w1 = gat1_sc(h4, atab, dtab, srcr, dstr)
    acc1, den1 = acc1[:, :n], den1[:n]

    m2, xa1 = pl.pallas_call(
        functools.partial(_tc_mid_body, nc=4),
        grid=(n // BN,),
        in_specs=[pl.BlockSpec((4, BN, CW), lambda i: (0, i, 0)),
                  _rows(16), _rep(16, 256), _vec(256),
                  _rep(256, 272), _rep(256, 80), _vec(80)],
        out_specs=[_rows(272), _rows(80)],
        out_shape=[jax.ShapeDtypeStruct((n, 272), jnp.float32),
                   jax.ShapeDtypeStruct((n, 80), jnp.float32)],
    )(acc1, den1, exp8, p["c1_b"], w2, p["a1_w"], p["a1_b"])

    h4, atab, dtab = _split_m(m2, 8, 32)
    acc2, den2, _w2 = gat1_sc(h4, atab, dtab, srcr, dstr)
    acc2, den2 = acc2[:, :n], den2[:n]

    m3, xa2 = pl.pallas_call(
        functools.partial(_tc_mid_body, nc=4),
        grid=(n // BN,),
        in_specs=[pl.BlockSpec((4, BN, CW), lambda i: (0, i, 0)),
                  _rows(16), _rep(16, 256), _vec(256),
                  _rep(256, 792), _rep(256, 200), _vec(200)],
        out_specs=[_rows(792), _rows(200)],
        out_shape=[jax.ShapeDtypeStruct((n, 792), jnp.float32),
                   jax.ShapeDtypeStruct((n, 200), jnp.float32)],
    )(acc2, den2, exp8, p["c2_b"], w3, p["a3_w"], p["a3_b"])

    h4, atab, dtab = _split_m(m3, 12, 64)
    acc3, den3, _w3 = gat3_sc(h4, atab, dtab, srcr, dstr)
    acc3, den3 = acc3[:, :n], den3[:n]

    f1 = p["f1_w"]
    out = pl.pallas_call(
        functools.partial(_tc_head_body, nc=12),
        grid=(n // BN,),
        in_specs=[pl.BlockSpec((12, BN, CW), lambda i: (0, i, 0)),
                  _rows(16), _rep(16, 768), _vec(768),
                  _rows(32), _rows(80), _rows(200),
                  _rep(768, 200), _rep(32, 200), _rep(80, 200),
                  _rep(200, 200), _vec(200),
                  _rep(200, 64), _vec(64), _rep(64, 1), _vec(1)],
        out_specs=_rows(1),
        out_shape=jax.ShapeDtypeStruct((n, 1), jnp.float32),
    )(acc3, den3, exp12, p["c3_b"], gg2, xa1, xa2,
      f1[32:800], f1[:32], f1[800:880], f1[880:1080], p["f1_b"],
      p["f2_w"], p["f2_b"], p["f3_w"], p["f3_b"])
    return out


# speedup vs baseline: 23.0329x; 1.1393x over previous
import functools

import jax
import jax.numpy as jnp
from jax import lax
from jax.experimental import pallas as pl
from jax.experimental.pallas import tpu as pltpu
from jax.experimental.pallas import tpu_sc as plsc

N_NODES = 10000
N_PAD = 10240
E_REAL = 330000
KB = 128
NB = 168
PT = NB * KB
E_PAD = 16 * PT
ROWS_T = N_PAD // 16
BN = 1000


def _mm_body(x_ref, w_ref, b_ref, o_ref, *, act):
    acc = jnp.dot(x_ref[...], w_ref[...], preferred_element_type=jnp.float32)
    acc = acc + b_ref[...][None, :]
    if act == "relu":
        acc = jnp.maximum(acc, 0.0)
    o_ref[...] = acc


def _mm(x, w, b, act="none"):
    n, k = x.shape
    m = w.shape[1]
    return pl.pallas_call(
        functools.partial(_mm_body, act=act),
        grid=(n // BN,),
        in_specs=[
            pl.BlockSpec((BN, k), lambda i: (i, 0)),
            pl.BlockSpec((k, m), lambda i: (0, 0)),
            pl.BlockSpec((m,), lambda i: (0,)),
        ],
        out_specs=pl.BlockSpec((BN, m), lambda i: (i, 0)),
        out_shape=jax.ShapeDtypeStruct((n, m), jnp.float32),
    )(x, w, b)


def _tc1_body(x_ref, w1_ref, ln1w_ref, ln1b_ref, ln2w_ref, ln2b_ref,
              m1_ref, gg2_ref):
    x = x_ref[...]
    m1_ref[...] = jnp.dot(x, w1_ref[...], preferred_element_type=jnp.float32)
    gg1 = jnp.maximum(jnp.dot(x, ln1w_ref[...],
                              preferred_element_type=jnp.float32)
                      + ln1b_ref[...][None, :], 0.0)
    gg2_ref[...] = jnp.maximum(jnp.dot(gg1, ln2w_ref[...],
                                       preferred_element_type=jnp.float32)
                               + ln2b_ref[...][None, :], 0.0)


def _tc_mid_body(acc_ref, den_ref, exp_ref, bias_ref, w_ref, wa_ref, ba_ref,
                 m_ref, xa_ref, *, nc):
    acc = jnp.concatenate([acc_ref[i] for i in range(nc)], axis=-1)
    dd = jnp.dot(den_ref[...], exp_ref[...],
                 preferred_element_type=jnp.float32)
    xl = jnp.maximum(acc / dd + bias_ref[...][None, :], 0.0)
    m_ref[...] = jnp.dot(xl, w_ref[...], preferred_element_type=jnp.float32)
    xa_ref[...] = jnp.maximum(jnp.dot(xl, wa_ref[...],
                                      preferred_element_type=jnp.float32)
                              + ba_ref[...][None, :], 0.0)


def _tc_head_body(acc_ref, den_ref, exp_ref, bias_ref, gg2_ref, xa1_ref,
                  xa2_ref, f1x_ref, f1g_ref, f1a1_ref, f1a2_ref, f1b_ref,
                  f2w_ref, f2b_ref, f3w_ref, f3b_ref, o_ref, *, nc):
    acc = jnp.concatenate([acc_ref[i] for i in range(nc)], axis=-1)
    dd = jnp.dot(den_ref[...], exp_ref[...],
                 preferred_element_type=jnp.float32)
    x3 = jnp.maximum(acc / dd + bias_ref[...][None, :], 0.0)
    xf = (jnp.dot(x3, f1x_ref[...], preferred_element_type=jnp.float32)
          + jnp.dot(gg2_ref[...], f1g_ref[...],
                    preferred_element_type=jnp.float32)
          + jnp.dot(xa1_ref[...], f1a1_ref[...],
                    preferred_element_type=jnp.float32)
          + jnp.dot(xa2_ref[...], f1a2_ref[...],
                    preferred_element_type=jnp.float32)
          + f1b_ref[...][None, :])
    xf = jnp.maximum(xf, 0.0)
    xf = jnp.maximum(jnp.dot(xf, f2w_ref[...],
                             preferred_element_type=jnp.float32)
                     + f2b_ref[...][None, :], 0.0)
    xf = jnp.dot(xf, f3w_ref[...], preferred_element_type=jnp.float32) \
        + f3b_ref[...][None, :]
    o_ref[...] = jax.nn.sigmoid(xf)


def _rep(k, m):
    return pl.BlockSpec((k, m), lambda i: (0, 0))


def _vec(m):
    return pl.BlockSpec((m,), lambda i: (0,))


def _rows(m):
    return pl.BlockSpec((BN, m), lambda i: (i, 0))


CW = 64
NV = CW // 16


@functools.lru_cache(maxsize=None)
def _make_gat_sc(heads, ch):
    shift = 5 if ch == 32 else 6
    nc = heads * ch // CW
    n_rounds = nc // 2
    n_al = CW // ch if ch < CW else 1
    mesh = plsc.VectorSubcoreMesh(core_axis_name="c", subcore_axis_name="s",
                                  num_cores=2, num_subcores=16)

    def body(h4, atab, dtab, srcr, dstr, acc_hbm, den_hbm, w_hbm,
             srcb, dstb, asb, adb, wb, hbuf, obuf, ridx, zb, zbd,
             acc, den, sems):
        g = lax.axis_index("c")
        sid = lax.axis_index("s")
        rowblk = sid * NB

        zv = jnp.zeros((16,), jnp.float32)

        def z1(i, carry):
            for v in range(NV):
                zb[i, pl.ds(v * 16, 16)] = zv
            return carry

        lax.fori_loop(0, 128, z1, 0)

        def z2(i, carry):
            zbd[i] = zv
            return carry

        lax.fori_loop(0, ROWS_T, z2, 0)

        nrow0 = sid * ROWS_T
        pltpu.sync_copy(zbd, den.at[pl.ds(nrow0, ROWS_T)])

        ebase0 = sid * PT
        sem_a, sem_d, sem_h, sem_ws, sem_ww, sem_ha, sem_e = sems

        def drain(kind, sl):
            if kind == "a":
                pltpu.make_async_copy(atab.at[pl.ds(0, KB)], asb.at[sl],
                                      sem_a.at[sl]).wait()
            elif kind == "d":
                pltpu.make_async_copy(dtab.at[pl.ds(0, KB)], adb.at[sl],
                                      sem_d.at[sl]).wait()
            elif kind == "h":
                pltpu.make_async_copy(h4.at[pl.ds(0, KB)], hbuf.at[sl],
                                      sem_h.at[sl]).wait()
            elif kind == "ws":
                pltpu.make_async_copy(wb.at[sl], den.at[pl.ds(0, KB)],
                                      sem_ws.at[sl]).wait()
            elif kind == "ww":
                pltpu.make_async_copy(wb.at[sl], w_hbm.at[pl.ds(0, KB)],
                                      sem_ww.at[sl]).wait()
            elif kind == "ha":
                pltpu.make_async_copy(obuf.at[sl], acc.at[pl.ds(0, KB)],
                                      sem_ha.at[sl]).wait()
            elif kind == "e":
                pltpu.make_async_copy(srcr.at[0], srcb.at[sl],
                                      sem_e.at[sl]).wait()
                pltpu.make_async_copy(srcr.at[0], srcb.at[sl],
                                      sem_e.at[sl]).wait()

        def pf_edges(b):
            el = jnp.bitwise_and(b, 7)
            pltpu.async_copy(srcr.at[rowblk + b], srcb.at[el], sem_e.at[el])
            pltpu.async_copy(dstr.at[rowblk + b], dstb.at[el], sem_e.at[el])

        for r in range(n_rounds):
            c = g * n_rounds + r
            for s5 in range(5):
                pltpu.sync_copy(zb, acc.at[pl.ds(nrow0 + s5 * 128, 128)])
            plsc.subcore_barrier()

            hgs = [jnp.full((16,), lax.shift_right_logical(
                c * CW + a * ch, shift), jnp.int32) for a in range(n_al)]

            def pf_gather(b, sl):
                el = jnp.bitwise_and(b, 7)
                drain("e", el)
                if r == 0:
                    pltpu.async_copy(atab.at[srcb.at[el]], asb.at[sl],
                                     sem_a.at[sl])
                    pltpu.async_copy(dtab.at[dstb.at[el]], adb.at[sl],
                                     sem_d.at[sl])
                else:
                    pltpu.async_copy(
                        w_hbm.at[pl.ds(ebase0 + b * KB, KB)], wb.at[sl],
                        sem_a.at[sl])

                @plsc.parallel_loop(0, KB, 16, unroll=2)
                def rix(i):
                    sv = srcb[el, pl.ds(i, 16)]
                    ridx[sl, pl.ds(i, 16)] = sv * nc + c
                pltpu.async_copy(h4.at[ridx.at[sl]], hbuf.at[sl],
                                 sem_h.at[sl])

            pf_edges(0)
            pf_edges(1)
            pf_edges(2)
            pf_gather(0, 0)
            pf_gather(1, 1)

            def section(j, b, sl):
                el = jnp.bitwise_and(b, 7)
                not_first = j > 0
                drain("a", sl)
                if r == 0:
                    drain("d", sl)
                drain("h", sl)

                @pl.when(not_first)
                def _drains():
                    if r == 0:
                        drain("ws", sl)
                        drain("ww", sl)
                    drain("ha", sl)

                if r == 0:
                    ebase = ebase0 + b * KB

                    @plsc.parallel_loop(0, KB, 1, unroll=4)
                    def edge(e):
                        ev = asb[sl, e] + adb[sl, e]
                        ev = jnp.where(ev >= 0.0, ev, 0.2 * ev)
                        scale = jnp.where(ebase + e < E_REAL, 1.0, 0.0)
                        wv = jnp.exp(ev) * scale
                        wb[sl, e] = wv
                        for v in range(NV):
                            al = wv.at[hgs[(v * 16) // ch]].get(
                                mode="promise_in_bounds")
                            obuf[sl, e, pl.ds(v * 16, 16)] = (
                                hbuf[sl, e, pl.ds(v * 16, 16)] * al)
                    pltpu.async_copy(wb.at[sl], den.at[dstb.at[el]],
                                     sem_ws.at[sl], add=True)
                    pltpu.async_copy(wb.at[sl],
                                     w_hbm.at[pl.ds(ebase0 + b * KB, KB)],
                                     sem_ww.at[sl])
                else:
                    @plsc.parallel_loop(0, KB, 1, unroll=4)
                    def edge(e):
                        wv = wb[sl, e]
                        for v in range(NV):
                            al = wv.at[hgs[(v * 16) // ch]].get(
                                mode="promise_in_bounds")
                            obuf[sl, e, pl.ds(v * 16, 16)] = (
                                hbuf[sl, e, pl.ds(v * 16, 16)] * al)

                pltpu.async_copy(obuf.at[sl], acc.at[dstb.at[el]],
                                 sem_ha.at[sl], add=True)

                @pl.when(b + 3 < NB)
                def _pfe():
                    pf_edges(b + 3)

                @pl.when(b + 2 < NB)
                def _pf():
                    pf_gather(b + 2, sl)

            def blk2(j, carry):
                section(j, 2 * j, 0)
                section(j, 2 * j + 1, 1)
                return carry

            lax.fori_loop(0, NB // 2, blk2, 0)

            for sl in range(2):
                if r == 0:
                    drain("ws", sl)
                    drain("ww", sl)
                drain("ha", sl)
            plsc.subcore_barrier()
            pltpu.sync_copy(acc.at[pl.ds(nrow0, ROWS_T)],
                            acc_hbm.at[c, pl.ds(nrow0, ROWS_T)])
            if r == 0:
                @pl.when(g == 0)
                def _dump_den():
                    pltpu.sync_copy(den.at[pl.ds(nrow0, ROWS_T)],
                                    den_hbm.at[pl.ds(nrow0, ROWS_T)])

    return pl.kernel(
        body,
        out_type=(
            jax.ShapeDtypeStruct((nc, N_PAD, CW), jnp.float32),
            jax.ShapeDtypeStruct((N_PAD, 16), jnp.float32),
            jax.ShapeDtypeStruct((E_PAD, 16), jnp.float32),
        ),
        mesh=mesh,
        scratch_types=[
            pltpu.VMEM((8, KB), jnp.int32),
            pltpu.VMEM((8, KB), jnp.int32),
            pltpu.VMEM((2, KB, 16), jnp.float32),
            pltpu.VMEM((2, KB, 16), jnp.float32),
            pltpu.VMEM((2, KB, 16), jnp.float32),
            pltpu.VMEM((2, KB, CW), jnp.float32),
            pltpu.VMEM((2, KB, CW), jnp.float32),
            pltpu.VMEM((2, KB), jnp.int32),
            pltpu.VMEM((128, CW), jnp.float32),
            pltpu.VMEM((ROWS_T, 16), jnp.float32),
            pltpu.VMEM_SHARED((N_PAD, CW), jnp.float32),
            pltpu.VMEM_SHARED((N_PAD, 16), jnp.float32),
            [pltpu.SemaphoreType.DMA((2,))] * 6
            + [pltpu.SemaphoreType.DMA((8,))],
        ],
        compiler_params=pltpu.CompilerParams(use_tc_tiling_on_sc=False),
    )


def _fold_attn(w, a_src, a_dst, heads, out_ch):
    wr = w.reshape(w.shape[0], heads, out_ch)
    ws = (wr * a_src[None]).sum(-1)
    wd = (wr * a_dst[None]).sum(-1)
    return jnp.concatenate([w, ws, wd], axis=1)


def _expand_mat(heads, ch):
    e = jnp.zeros((16, heads * ch), jnp.float32)
    r = jnp.arange(heads * ch) // ch
    e = e.at[r, jnp.arange(heads * ch)].set(1.0)
    return e


def _split_m(m, heads, ch):
    hc = heads * ch
    nc = hc // CW
    h4 = m[:, :hc].reshape(N_NODES * nc, CW)
    atab = jnp.pad(m[:, hc:hc + heads], ((0, 0), (0, 16 - heads)))
    dtab = jnp.pad(m[:, hc + heads:hc + 2 * heads], ((0, 0), (0, 16 - heads)))
    return h4, atab, dtab


def kernel(x, params, edge_index):
    p = params
    n = N_NODES
    loop = jnp.arange(n, dtype=edge_index.dtype)
    padz = jnp.zeros((E_PAD - E_REAL,), edge_index.dtype)
    srcr = jnp.concatenate([edge_index[0], loop, padz]).reshape(16 * NB, KB)
    dstr = jnp.concatenate([edge_index[1], loop, padz]).reshape(16 * NB, KB)

    w1 = _fold_attn(p["c1_w"], p["c1_as"], p["c1_ad"], 8, 32)
    w2 = _fold_attn(p["c2_w"], p["c2_as"], p["c2_ad"], 8, 32)
    w3 = _fold_attn(p["c3_w"], p["c3_as"], p["c3_ad"], 12, 64)
    exp8 = _expand_mat(8, 32)
    exp12 = _expand_mat(12, 64)

    m1, gg2 = pl.pallas_call(
        _tc1_body,
        grid=(n // BN,),
        in_specs=[_rows(128), _rep(128, 272), _rep(128, 32), _vec(32),
                  _rep(32, 32), _vec(32)],
        out_specs=[_rows(272), _rows(32)],
        out_shape=[jax.ShapeDtypeStruct((n, 272), jnp.float32),
                   jax.ShapeDtypeStruct((n, 32), jnp.float32)],
    )(x, w1, p["ln1_w"], p["ln1_b"], p["ln2_w"], p["ln2_b"])

    gat1_sc = _make_gat_sc(8, 32)
    gat3_sc = _make_gat_sc(12, 64)

    h4, atab, dtab = _split_m(m1, 8, 32)
    acc1, den1, _w1 = gat1_sc(h4, atab, dtab, srcr, dstr)
    acc1, den1 = acc1[:, :n], den1[:n]

    m2, xa1 = pl.pallas_call(
        functools.partial(_tc_mid_body, nc=4),
        grid=(n // BN,),
        in_specs=[pl.BlockSpec((4, BN, CW), lambda i: (0, i, 0)),
                  _rows(16), _rep(16, 256), _vec(256),
                  _rep(256, 272), _rep(256, 80), _vec(80)],
        out_specs=[_rows(272), _rows(80)],
        out_shape=[jax.ShapeDtypeStruct((n, 272), jnp.float32),
                   jax.ShapeDtypeStruct((n, 80), jnp.float32)],
    )(acc1, den1, exp8, p["c1_b"], w2, p["a1_w"], p["a1_b"])

    h4, atab, dtab = _split_m(m2, 8, 32)
    acc2, den2, _w2 = gat1_sc(h4, atab, dtab, srcr, dstr)
    acc2, den2 = acc2[:, :n], den2[:n]

    m3, xa2 = pl.pallas_call(
        functools.partial(_tc_mid_body, nc=4),
        grid=(n // BN,),
        in_specs=[pl.BlockSpec((4, BN, CW), lambda i: (0, i, 0)),
                  _rows(16), _rep(16, 256), _vec(256),
                  _rep(256, 792), _rep(256, 200), _vec(200)],
        out_specs=[_rows(792), _rows(200)],
        out_shape=[jax.ShapeDtypeStruct((n, 792), jnp.float32),
                   jax.ShapeDtypeStruct((n, 200), jnp.float32)],
    )(acc2, den2, exp8, p["c2_b"], w3, p["a3_w"], p["a3_b"])

    h4, atab, dtab = _split_m(m3, 12, 64)
    acc3, den3, _w3 = gat3_sc(h4, atab, dtab, srcr, dstr)
    acc3, den3 = acc3[:, :n], den3[:n]

    f1 = p["f1_w"]
    out = pl.pallas_call(
        functools.partial(_tc_head_body, nc=12),
        grid=(n // BN,),
        in_specs=[pl.BlockSpec((12, BN, CW), lambda i: (0, i, 0)),
                  _rows(16), _rep(16, 768), _vec(768),
                  _rows(32), _rows(80), _rows(200),
                  _rep(768, 200), _rep(32, 200), _rep(80, 200),
                  _rep(200, 200), _vec(200),
                  _rep(200, 64), _vec(64), _rep(64, 1), _vec(1)],
        out_specs=_rows(1),
        out_shape=jax.ShapeDtypeStruct((n, 1), jnp.float32),
    )(acc3, den3, exp12, p["c3_b"], gg2, xa1, xa2,
      f1[32:800], f1[:32], f1[800:880], f1[880:1080], p["f1_b"],
      p["f2_w"], p["f2_b"], p["f3_w"], p["f3_b"])
    return out


# speedup vs baseline: 23.0442x; 1.0005x over previous
import functools

import jax
import jax.numpy as jnp
from jax import lax
from jax.experimental import pallas as pl
from jax.experimental.pallas import tpu as pltpu
from jax.experimental.pallas import tpu_sc as plsc

N_NODES = 10000
N_PAD = 10240
E_REAL = 330000
KB = 128
NB = 168
PT = NB * KB
E_PAD = 16 * PT
ROWS_T = N_PAD // 16
BN = 1000


def _mm_body(x_ref, w_ref, b_ref, o_ref, *, act):
    acc = jnp.dot(x_ref[...], w_ref[...], preferred_element_type=jnp.float32)
    acc = acc + b_ref[...][None, :]
    if act == "relu":
        acc = jnp.maximum(acc, 0.0)
    o_ref[...] = acc


def _mm(x, w, b, act="none"):
    n, k = x.shape
    m = w.shape[1]
    return pl.pallas_call(
        functools.partial(_mm_body, act=act),
        grid=(n // BN,),
        in_specs=[
            pl.BlockSpec((BN, k), lambda i: (i, 0)),
            pl.BlockSpec((k, m), lambda i: (0, 0)),
            pl.BlockSpec((m,), lambda i: (0,)),
        ],
        out_specs=pl.BlockSpec((BN, m), lambda i: (i, 0)),
        out_shape=jax.ShapeDtypeStruct((n, m), jnp.float32),
    )(x, w, b)


def _tc1_body(x_ref, w1_ref, ln1w_ref, ln1b_ref, ln2w_ref, ln2b_ref,
              m1_ref, gg2_ref):
    x = x_ref[...]
    m1_ref[...] = jnp.dot(x, w1_ref[...], preferred_element_type=jnp.float32)
    gg1 = jnp.maximum(jnp.dot(x, ln1w_ref[...],
                              preferred_element_type=jnp.float32)
                      + ln1b_ref[...][None, :], 0.0)
    gg2_ref[...] = jnp.maximum(jnp.dot(gg1, ln2w_ref[...],
                                       preferred_element_type=jnp.float32)
                               + ln2b_ref[...][None, :], 0.0)


def _tc_mid_body(acc_ref, den_ref, exp_ref, bias_ref, w_ref, wa_ref, ba_ref,
                 m_ref, xa_ref, *, nc):
    acc = jnp.concatenate([acc_ref[i] for i in range(nc)], axis=-1)
    dd = jnp.dot(den_ref[...], exp_ref[...],
                 preferred_element_type=jnp.float32)
    xl = jnp.maximum(acc / dd + bias_ref[...][None, :], 0.0)
    m_ref[...] = jnp.dot(xl, w_ref[...], preferred_element_type=jnp.float32)
    xa_ref[...] = jnp.maximum(jnp.dot(xl, wa_ref[...],
                                      preferred_element_type=jnp.float32)
                              + ba_ref[...][None, :], 0.0)


def _tc_head_body(acc_ref, den_ref, exp_ref, bias_ref, gg2_ref, xa1_ref,
                  xa2_ref, f1x_ref, f1g_ref, f1a1_ref, f1a2_ref, f1b_ref,
                  f2w_ref, f2b_ref, f3w_ref, f3b_ref, o_ref, *, nc):
    acc = jnp.concatenate([acc_ref[i] for i in range(nc)], axis=-1)
    dd = jnp.dot(den_ref[...], exp_ref[...],
                 preferred_element_type=jnp.float32)
    x3 = jnp.maximum(acc / dd + bias_ref[...][None, :], 0.0)
    xf = (jnp.dot(x3, f1x_ref[...], preferred_element_type=jnp.float32)
          + jnp.dot(gg2_ref[...], f1g_ref[...],
                    preferred_element_type=jnp.float32)
          + jnp.dot(xa1_ref[...], f1a1_ref[...],
                    preferred_element_type=jnp.float32)
          + jnp.dot(xa2_ref[...], f1a2_ref[...],
                    preferred_element_type=jnp.float32)
          + f1b_ref[...][None, :])
    xf = jnp.maximum(xf, 0.0)
    xf = jnp.maximum(jnp.dot(xf, f2w_ref[...],
                             preferred_element_type=jnp.float32)
                     + f2b_ref[...][None, :], 0.0)
    xf = jnp.dot(xf, f3w_ref[...], preferred_element_type=jnp.float32) \
        + f3b_ref[...][None, :]
    o_ref[...] = jax.nn.sigmoid(xf)


def _rep(k, m):
    return pl.BlockSpec((k, m), lambda i: (0, 0))


def _vec(m):
    return pl.BlockSpec((m,), lambda i: (0,))


def _rows(m):
    return pl.BlockSpec((BN, m), lambda i: (i, 0))


CW = 64
NV = CW // 16


@functools.lru_cache(maxsize=None)
def _make_gat_sc(heads, ch):
    shift = 5 if ch == 32 else 6
    nc = heads * ch // CW
    n_rounds = nc // 2
    n_al = CW // ch if ch < CW else 1
    mesh = plsc.VectorSubcoreMesh(core_axis_name="c", subcore_axis_name="s",
                                  num_cores=2, num_subcores=16)

    def body(h4, atab, dtab, srcr, dstr, acc_hbm, den_hbm, w_hbm,
             srcb, dstb, asb, adb, wb, hbuf, obuf, ridx, zb, zbd,
             acc, den, sems):
        g = lax.axis_index("c")
        sid = lax.axis_index("s")
        rowblk = sid * NB

        zv = jnp.zeros((16,), jnp.float32)

        def z1(i, carry):
            for v in range(NV):
                zb[i, pl.ds(v * 16, 16)] = zv
            return carry

        lax.fori_loop(0, 128, z1, 0)

        def z2(i, carry):
            zbd[i] = zv
            return carry

        lax.fori_loop(0, ROWS_T, z2, 0)

        nrow0 = sid * ROWS_T
        pltpu.sync_copy(zbd, den.at[pl.ds(nrow0, ROWS_T)])

        ebase0 = sid * PT
        sem_a, sem_d, sem_h, sem_ws, sem_ww, sem_ha, sem_e = sems

        def drain(kind, sl):
            if kind == "a":
                pltpu.make_async_copy(atab.at[pl.ds(0, KB)], asb.at[sl],
                                      sem_a.at[sl]).wait()
            elif kind == "d":
                pltpu.make_async_copy(dtab.at[pl.ds(0, KB)], adb.at[sl],
                                      sem_d.at[sl]).wait()
            elif kind == "h":
                pltpu.make_async_copy(h4.at[pl.ds(0, KB)], hbuf.at[sl],
                                      sem_h.at[sl]).wait()
            elif kind == "ws":
                pltpu.make_async_copy(wb.at[sl], den.at[pl.ds(0, KB)],
                                      sem_ws.at[sl]).wait()
            elif kind == "ww":
                pltpu.make_async_copy(wb.at[sl], w_hbm.at[pl.ds(0, KB)],
                                      sem_ww.at[sl]).wait()
            elif kind == "ha":
                pltpu.make_async_copy(obuf.at[sl], acc.at[pl.ds(0, KB)],
                                      sem_ha.at[sl]).wait()
            elif kind == "e":
                pltpu.make_async_copy(srcr.at[0], srcb.at[sl],
                                      sem_e.at[sl]).wait()
                pltpu.make_async_copy(srcr.at[0], srcb.at[sl],
                                      sem_e.at[sl]).wait()

        def pf_edges(b):
            el = jnp.bitwise_and(b, 7)
            pltpu.async_copy(srcr.at[rowblk + b], srcb.at[el], sem_e.at[el])
            pltpu.async_copy(dstr.at[rowblk + b], dstb.at[el], sem_e.at[el])

        for r in range(n_rounds):
            c = g * n_rounds + r
            for s5 in range(5):
                pltpu.sync_copy(zb, acc.at[pl.ds(nrow0 + s5 * 128, 128)])
            plsc.subcore_barrier()

            hgs = [jnp.full((16,), lax.shift_right_logical(
                c * CW + a * ch, shift), jnp.int32) for a in range(n_al)]

            def pf_gather(b, sl):
                el = jnp.bitwise_and(b, 7)
                drain("e", el)
                if r == 0:
                    pltpu.async_copy(atab.at[srcb.at[el]], asb.at[sl],
                                     sem_a.at[sl])
                    pltpu.async_copy(dtab.at[dstb.at[el]], adb.at[sl],
                                     sem_d.at[sl])
                else:
                    pltpu.async_copy(
                        w_hbm.at[pl.ds(ebase0 + b * KB, KB)], wb.at[sl],
                        sem_a.at[sl])

                @plsc.parallel_loop(0, KB, 16, unroll=2)
                def rix(i):
                    sv = srcb[el, pl.ds(i, 16)]
                    ridx[sl, pl.ds(i, 16)] = sv * nc + c
                pltpu.async_copy(h4.at[ridx.at[sl]], hbuf.at[sl],
                                 sem_h.at[sl])

            pf_edges(0)
            pf_edges(1)
            pf_edges(2)
            pf_gather(0, 0)
            pf_gather(1, 1)

            def section(j, b, sl):
                el = jnp.bitwise_and(b, 7)
                not_first = j > 0
                drain("a", sl)
                if r == 0:
                    drain("d", sl)
                drain("h", sl)

                @pl.when(not_first)
                def _drains():
                    if r == 0:
                        drain("ws", sl)
                        drain("ww", sl)
                    drain("ha", sl)

                if r == 0:
                    ebase = ebase0 + b * KB

                    @plsc.parallel_loop(0, KB, 1, unroll=4)
                    def edge(e):
                        ev = asb[sl, e] + adb[sl, e]
                        ev = jnp.where(ev >= 0.0, ev, 0.2 * ev)
                        scale = jnp.where(ebase + e < E_REAL, 1.0, 0.0)
                        wv = jnp.exp(ev) * scale
                        wb[sl, e] = wv
                        for v in range(NV):
                            al = wv.at[hgs[(v * 16) // ch]].get(
                                mode="promise_in_bounds")
                            obuf[sl, e, pl.ds(v * 16, 16)] = (
                                hbuf[sl, e, pl.ds(v * 16, 16)] * al)
                    pltpu.async_copy(wb.at[sl], den.at[dstb.at[el]],
                                     sem_ws.at[sl], add=True)
                    pltpu.async_copy(wb.at[sl],
                                     w_hbm.at[pl.ds(ebase0 + b * KB, KB)],
                                     sem_ww.at[sl])
                else:
                    @plsc.parallel_loop(0, KB, 1, unroll=8)
                    def edge(e):
                        wv = wb[sl, e]
                        for v in range(NV):
                            al = wv.at[hgs[(v * 16) // ch]].get(
                                mode="promise_in_bounds")
                            obuf[sl, e, pl.ds(v * 16, 16)] = (
                                hbuf[sl, e, pl.ds(v * 16, 16)] * al)

                pltpu.async_copy(obuf.at[sl], acc.at[dstb.at[el]],
                                 sem_ha.at[sl], add=True)

                @pl.when(b + 3 < NB)
                def _pfe():
                    pf_edges(b + 3)

                @pl.when(b + 2 < NB)
                def _pf():
                    pf_gather(b + 2, sl)

            def blk2(j, carry):
                section(j, 2 * j, 0)
                section(j, 2 * j + 1, 1)
                return carry

            lax.fori_loop(0, NB // 2, blk2, 0)

            for sl in range(2):
                if r == 0:
                    drain("ws", sl)
                    drain("ww", sl)
                drain("ha", sl)
            plsc.subcore_barrier()
            pltpu.sync_copy(acc.at[pl.ds(nrow0, ROWS_T)],
                            acc_hbm.at[c, pl.ds(nrow0, ROWS_T)])
            if r == 0:
                @pl.when(g == 0)
                def _dump_den():
                    pltpu.sync_copy(den.at[pl.ds(nrow0, ROWS_T)],
                                    den_hbm.at[pl.ds(nrow0, ROWS_T)])

    return pl.kernel(
        body,
        out_type=(
            jax.ShapeDtypeStruct((nc, N_PAD, CW), jnp.float32),
            jax.ShapeDtypeStruct((N_PAD, 16), jnp.float32),
            jax.ShapeDtypeStruct((E_PAD, 16), jnp.float32),
        ),
        mesh=mesh,
        scratch_types=[
            pltpu.VMEM((8, KB), jnp.int32),
            pltpu.VMEM((8, KB), jnp.int32),
            pltpu.VMEM((2, KB, 16), jnp.float32),
            pltpu.VMEM((2, KB, 16), jnp.float32),
            pltpu.VMEM((2, KB, 16), jnp.float32),
            pltpu.VMEM((2, KB, CW), jnp.float32),
            pltpu.VMEM((2, KB, CW), jnp.float32),
            pltpu.VMEM((2, KB), jnp.int32),
            pltpu.VMEM((128, CW), jnp.float32),
            pltpu.VMEM((ROWS_T, 16), jnp.float32),
            pltpu.VMEM_SHARED((N_PAD, CW), jnp.float32),
            pltpu.VMEM_SHARED((N_PAD, 16), jnp.float32),
            [pltpu.SemaphoreType.DMA((2,))] * 6
            + [pltpu.SemaphoreType.DMA((8,))],
        ],
        compiler_params=pltpu.CompilerParams(use_tc_tiling_on_sc=False),
    )


def _fold_attn(w, a_src, a_dst, heads, out_ch):
    wr = w.reshape(w.shape[0], heads, out_ch)
    ws = (wr * a_src[None]).sum(-1)
    wd = (wr * a_dst[None]).sum(-1)
    return jnp.concatenate([w, ws, wd], axis=1)


def _expand_mat(heads, ch):
    e = jnp.zeros((16, heads * ch), jnp.float32)
    r = jnp.arange(heads * ch) // ch
    e = e.at[r, jnp.arange(heads * ch)].set(1.0)
    return e


def _split_m(m, heads, ch):
    hc = heads * ch
    nc = hc // CW
    h4 = m[:, :hc].reshape(N_NODES * nc, CW)
    atab = jnp.pad(m[:, hc:hc + heads], ((0, 0), (0, 16 - heads)))
    dtab = jnp.pad(m[:, hc + heads:hc + 2 * heads], ((0, 0), (0, 16 - heads)))
    return h4, atab, dtab


def kernel(x, params, edge_index):
    p = params
    n = N_NODES
    loop = jnp.arange(n, dtype=edge_index.dtype)
    padz = jnp.zeros((E_PAD - E_REAL,), edge_index.dtype)
    srcr = jnp.concatenate([edge_index[0], loop, padz]).reshape(16 * NB, KB)
    dstr = jnp.concatenate([edge_index[1], loop, padz]).reshape(16 * NB, KB)

    w1 = _fold_attn(p["c1_w"], p["c1_as"], p["c1_ad"], 8, 32)
    w2 = _fold_attn(p["c2_w"], p["c2_as"], p["c2_ad"], 8, 32)
    w3 = _fold_attn(p["c3_w"], p["c3_as"], p["c3_ad"], 12, 64)
    exp8 = _expand_mat(8, 32)
    exp12 = _expand_mat(12, 64)

    m1, gg2 = pl.pallas_call(
        _tc1_body,
        grid=(n // BN,),
        in_specs=[_rows(128), _rep(128, 272), _rep(128, 32), _vec(32),
                  _rep(32, 32), _vec(32)],
        out_specs=[_rows(272), _rows(32)],
        out_shape=[jax.ShapeDtypeStruct((n, 272), jnp.float32),
                   jax.ShapeDtypeStruct((n, 32), jnp.float32)],
    )(x, w1, p["ln1_w"], p["ln1_b"], p["ln2_w"], p["ln2_b"])

    gat1_sc = _make_gat_sc(8, 32)
    gat3_sc = _make_gat_sc(12, 64)

    h4, atab, dtab = _split_m(m1, 8, 32)
    acc1, den1, _w1 = gat1_sc(h4, atab, dtab, srcr, dstr)
    acc1, den1 = acc1[:, :n], den1[:n]

    m2, xa1 = pl.pallas_call(
        functools.partial(_tc_mid_body, nc=4),
        grid=(n // BN,),
        in_specs=[pl.BlockSpec((4, BN, CW), lambda i: (0, i, 0)),
                  _rows(16), _rep(16, 256), _vec(256),
                  _rep(256, 272), _rep(256, 80), _vec(80)],
        out_specs=[_rows(272), _rows(80)],
        out_shape=[jax.ShapeDtypeStruct((n, 272), jnp.float32),
                   jax.ShapeDtypeStruct((n, 80), jnp.float32)],
    )(acc1, den1, exp8, p["c1_b"], w2, p["a1_w"], p["a1_b"])

    h4, atab, dtab = _split_m(m2, 8, 32)
    acc2, den2, _w2 = gat1_sc(h4, atab, dtab, srcr, dstr)
    acc2, den2 = acc2[:, :n], den2[:n]

    m3, xa2 = pl.pallas_call(
        functools.partial(_tc_mid_body, nc=4),
        grid=(n // BN,),
        in_specs=[pl.BlockSpec((4, BN, CW), lambda i: (0, i, 0)),
                  _rows(16), _rep(16, 256), _vec(256),
                  _rep(256, 792), _rep(256, 200), _vec(200)],
        out_specs=[_rows(792), _rows(200)],
        out_shape=[jax.ShapeDtypeStruct((n, 792), jnp.float32),
                   jax.ShapeDtypeStruct((n, 200), jnp.float32)],
    )(acc2, den2, exp8, p["c2_b"], w3, p["a3_w"], p["a3_b"])

    h4, atab, dtab = _split_m(m3, 12, 64)
    acc3, den3, _w3 = gat3_sc(h4, atab, dtab, srcr, dstr)
    acc3, den3 = acc3[:, :n], den3[:n]

    f1 = p["f1_w"]
    out = pl.pallas_call(
        functools.partial(_tc_head_body, nc=12),
        grid=(n // BN,),
        in_specs=[pl.BlockSpec((12, BN, CW), lambda i: (0, i, 0)),
                  _rows(16), _rep(16, 768), _vec(768),
                  _rows(32), _rows(80), _rows(200),
                  _rep(768, 200), _rep(32, 200), _rep(80, 200),
                  _rep(200, 200), _vec(200),
                  _rep(200, 64), _vec(64), _rep(64, 1), _vec(1)],
        out_specs=_rows(1),
        out_shape=jax.ShapeDtypeStruct((n, 1), jnp.float32),
    )(acc3, den3, exp12, p["c3_b"], gg2, xa1, xa2,
      f1[32:800], f1[:32], f1[800:880], f1[880:1080], p["f1_b"],
      p["f2_w"], p["f2_b"], p["f3_w"], p["f3_b"])
    return out


# speedup vs baseline: 23.7806x; 1.0320x over previous
import functools

import jax
import jax.numpy as jnp
from jax import lax
from jax.experimental import pallas as pl
from jax.experimental.pallas import tpu as pltpu
from jax.experimental.pallas import tpu_sc as plsc

N_NODES = 10000
N_PAD = 10240
E_REAL = 330000
KB = 128
NB = 168
PT = NB * KB
E_PAD = 16 * PT
ROWS_T = N_PAD // 16
BN = 1000


def _mm_body(x_ref, w_ref, b_ref, o_ref, *, act):
    acc = jnp.dot(x_ref[...], w_ref[...], preferred_element_type=jnp.float32)
    acc = acc + b_ref[...][None, :]
    if act == "relu":
        acc = jnp.maximum(acc, 0.0)
    o_ref[...] = acc


def _mm(x, w, b, act="none"):
    n, k = x.shape
    m = w.shape[1]
    return pl.pallas_call(
        functools.partial(_mm_body, act=act),
        grid=(n // BN,),
        in_specs=[
            pl.BlockSpec((BN, k), lambda i: (i, 0)),
            pl.BlockSpec((k, m), lambda i: (0, 0)),
            pl.BlockSpec((m,), lambda i: (0,)),
        ],
        out_specs=pl.BlockSpec((BN, m), lambda i: (i, 0)),
        out_shape=jax.ShapeDtypeStruct((n, m), jnp.float32),
    )(x, w, b)


def _tc1_body(x_ref, w1_ref, ln1w_ref, ln1b_ref, ln2w_ref, ln2b_ref,
              m1_ref, gg2_ref):
    x = x_ref[...]
    m1_ref[...] = jnp.dot(x, w1_ref[...], preferred_element_type=jnp.float32)
    gg1 = jnp.maximum(jnp.dot(x, ln1w_ref[...],
                              preferred_element_type=jnp.float32)
                      + ln1b_ref[...][None, :], 0.0)
    gg2_ref[...] = jnp.maximum(jnp.dot(gg1, ln2w_ref[...],
                                       preferred_element_type=jnp.float32)
                               + ln2b_ref[...][None, :], 0.0)


def _tc_mid_body(acc_ref, den_ref, exp_ref, bias_ref, w_ref, wa_ref, ba_ref,
                 m_ref, xa_ref, *, nc):
    acc = jnp.concatenate([acc_ref[i] for i in range(nc)], axis=-1)
    dd = jnp.dot(den_ref[...], exp_ref[...],
                 preferred_element_type=jnp.float32)
    xl = jnp.maximum(acc / dd + bias_ref[...][None, :], 0.0)
    m_ref[...] = jnp.dot(xl, w_ref[...], preferred_element_type=jnp.float32)
    xa_ref[...] = jnp.maximum(jnp.dot(xl, wa_ref[...],
                                      preferred_element_type=jnp.float32)
                              + ba_ref[...][None, :], 0.0)


def _tc_head_body(acc_ref, den_ref, exp_ref, bias_ref, gg2_ref, xa1_ref,
                  xa2_ref, f1x_ref, f1g_ref, f1a1_ref, f1a2_ref, f1b_ref,
                  f2w_ref, f2b_ref, f3w_ref, f3b_ref, o_ref, *, nc):
    acc = jnp.concatenate([acc_ref[i] for i in range(nc)], axis=-1)
    dd = jnp.dot(den_ref[...], exp_ref[...],
                 preferred_element_type=jnp.float32)
    x3 = jnp.maximum(acc / dd + bias_ref[...][None, :], 0.0)
    xf = (jnp.dot(x3, f1x_ref[...], preferred_element_type=jnp.float32)
          + jnp.dot(gg2_ref[...], f1g_ref[...],
                    preferred_element_type=jnp.float32)
          + jnp.dot(xa1_ref[...], f1a1_ref[...],
                    preferred_element_type=jnp.float32)
          + jnp.dot(xa2_ref[...], f1a2_ref[...],
                    preferred_element_type=jnp.float32)
          + f1b_ref[...][None, :])
    xf = jnp.maximum(xf, 0.0)
    xf = jnp.maximum(jnp.dot(xf, f2w_ref[...],
                             preferred_element_type=jnp.float32)
                     + f2b_ref[...][None, :], 0.0)
    xf = jnp.dot(xf, f3w_ref[...], preferred_element_type=jnp.float32) \
        + f3b_ref[...][None, :]
    o_ref[...] = jax.nn.sigmoid(xf)


def _rep(k, m):
    return pl.BlockSpec((k, m), lambda i: (0, 0))


def _vec(m):
    return pl.BlockSpec((m,), lambda i: (0,))


def _rows(m):
    return pl.BlockSpec((BN, m), lambda i: (i, 0))


CW = 64
NV = CW // 16


@functools.lru_cache(maxsize=None)
def _make_gat_sc(heads, ch):
    shift = 5 if ch == 32 else 6
    nc = heads * ch // CW
    n_rounds = nc // 2
    n_al = CW // ch if ch < CW else 1
    mesh = plsc.VectorSubcoreMesh(core_axis_name="c", subcore_axis_name="s",
                                  num_cores=2, num_subcores=16)

    def body(h4, atab, dtab, srcr, dstr, zac, zde, acc_hbm, den_hbm, w_hbm,
             srcb, dstb, asb, adb, wb, hbuf, obuf, ridx,
             acc, den, sems):
        g = lax.axis_index("c")
        sid = lax.axis_index("s")
        rowblk = sid * NB

        nrow0 = sid * ROWS_T
        pltpu.sync_copy(zde, den.at[pl.ds(nrow0, ROWS_T)])

        ebase0 = sid * PT
        sem_a, sem_d, sem_h, sem_ws, sem_ww, sem_ha, sem_e = sems

        def drain(kind, sl):
            if kind == "a":
                pltpu.make_async_copy(atab.at[pl.ds(0, KB)], asb.at[sl],
                                      sem_a.at[sl]).wait()
            elif kind == "d":
                pltpu.make_async_copy(dtab.at[pl.ds(0, KB)], adb.at[sl],
                                      sem_d.at[sl]).wait()
            elif kind == "h":
                pltpu.make_async_copy(h4.at[pl.ds(0, KB)], hbuf.at[sl],
                                      sem_h.at[sl]).wait()
            elif kind == "ws":
                pltpu.make_async_copy(wb.at[sl], den.at[pl.ds(0, KB)],
                                      sem_ws.at[sl]).wait()
            elif kind == "ww":
                pltpu.make_async_copy(wb.at[sl], w_hbm.at[pl.ds(0, KB)],
                                      sem_ww.at[sl]).wait()
            elif kind == "ha":
                pltpu.make_async_copy(obuf.at[sl], acc.at[pl.ds(0, KB)],
                                      sem_ha.at[sl]).wait()
            elif kind == "e":
                pltpu.make_async_copy(srcr.at[0], srcb.at[sl],
                                      sem_e.at[sl]).wait()
                pltpu.make_async_copy(srcr.at[0], srcb.at[sl],
                                      sem_e.at[sl]).wait()

        def pf_edges(b):
            el = jnp.bitwise_and(b, 7)
            pltpu.async_copy(srcr.at[rowblk + b], srcb.at[el], sem_e.at[el])
            pltpu.async_copy(dstr.at[rowblk + b], dstb.at[el], sem_e.at[el])

        for r in range(n_rounds):
            c = g * n_rounds + r
            pltpu.sync_copy(zac, acc.at[pl.ds(nrow0, ROWS_T)])
            plsc.subcore_barrier()

            hgs = [jnp.full((16,), lax.shift_right_logical(
                c * CW + a * ch, shift), jnp.int32) for a in range(n_al)]

            def pf_gather(b, sl):
                el = jnp.bitwise_and(b, 7)
                drain("e", el)
                if r == 0:
                    pltpu.async_copy(atab.at[srcb.at[el]], asb.at[sl],
                                     sem_a.at[sl])
                    pltpu.async_copy(dtab.at[dstb.at[el]], adb.at[sl],
                                     sem_d.at[sl])
                else:
                    pltpu.async_copy(
                        w_hbm.at[pl.ds(ebase0 + b * KB, KB)], wb.at[sl],
                        sem_a.at[sl])

                @plsc.parallel_loop(0, KB, 16, unroll=2)
                def rix(i):
                    sv = srcb[el, pl.ds(i, 16)]
                    ridx[sl, pl.ds(i, 16)] = sv * nc + c
                pltpu.async_copy(h4.at[ridx.at[sl]], hbuf.at[sl],
                                 sem_h.at[sl])

            pf_edges(0)
            pf_edges(1)
            pf_edges(2)
            pf_gather(0, jnp.int32(0))
            pf_gather(1, jnp.int32(1))

            def section(b, sl):
                el = jnp.bitwise_and(b, 7)
                not_first = b >= 3
                drain("a", sl)
                if r == 0:
                    drain("d", sl)
                drain("h", sl)

                @pl.when(not_first)
                def _drains():
                    if r == 0:
                        drain("ws", sl)
                        drain("ww", sl)
                    drain("ha", sl)

                if r == 0:
                    ebase = ebase0 + b * KB

                    @plsc.parallel_loop(0, KB, 1, unroll=4)
                    def edge(e):
                        ev = asb[sl, e] + adb[sl, e]
                        ev = jnp.where(ev >= 0.0, ev, 0.2 * ev)
                        scale = jnp.where(ebase + e < E_REAL, 1.0, 0.0)
                        wv = jnp.exp(ev) * scale
                        wb[sl, e] = wv
                        for v in range(NV):
                            al = wv.at[hgs[(v * 16) // ch]].get(
                                mode="promise_in_bounds")
                            obuf[sl, e, pl.ds(v * 16, 16)] = (
                                hbuf[sl, e, pl.ds(v * 16, 16)] * al)
                    pltpu.async_copy(wb.at[sl], den.at[dstb.at[el]],
                                     sem_ws.at[sl], add=True)
                    pltpu.async_copy(wb.at[sl],
                                     w_hbm.at[pl.ds(ebase0 + b * KB, KB)],
                                     sem_ww.at[sl])
                else:
                    @plsc.parallel_loop(0, KB, 1, unroll=8)
                    def edge(e):
                        wv = wb[sl, e]
                        for v in range(NV):
                            al = wv.at[hgs[(v * 16) // ch]].get(
                                mode="promise_in_bounds")
                            obuf[sl, e, pl.ds(v * 16, 16)] = (
                                hbuf[sl, e, pl.ds(v * 16, 16)] * al)

                pltpu.async_copy(obuf.at[sl], acc.at[dstb.at[el]],
                                 sem_ha.at[sl], add=True)

                @pl.when(b + 3 < NB)
                def _pfe():
                    pf_edges(b + 3)

                @pl.when(b + 2 < NB)
                def _pf():
                    pf_gather(b + 2, lax.rem(b + 2, jnp.int32(3)))

            def blk(b, carry):
                section(b, lax.rem(b, jnp.int32(3)))
                return carry

            lax.fori_loop(0, NB, blk, 0)

            for sl in range(3):
                if r == 0:
                    drain("ws", sl)
                    drain("ww", sl)
                drain("ha", sl)
            plsc.subcore_barrier()
            pltpu.sync_copy(acc.at[pl.ds(nrow0, ROWS_T)],
                            acc_hbm.at[c, pl.ds(nrow0, ROWS_T)])
            if r == 0:
                @pl.when(g == 0)
                def _dump_den():
                    pltpu.sync_copy(den.at[pl.ds(nrow0, ROWS_T)],
                                    den_hbm.at[pl.ds(nrow0, ROWS_T)])

    return pl.kernel(
        body,
        out_type=(
            jax.ShapeDtypeStruct((nc, N_PAD, CW), jnp.float32),
            jax.ShapeDtypeStruct((N_PAD, 16), jnp.float32),
            jax.ShapeDtypeStruct((E_PAD, 16), jnp.float32),
        ),
        mesh=mesh,
        scratch_types=[
            pltpu.VMEM((8, KB), jnp.int32),
            pltpu.VMEM((8, KB), jnp.int32),
            pltpu.VMEM((3, KB, 16), jnp.float32),
            pltpu.VMEM((3, KB, 16), jnp.float32),
            pltpu.VMEM((3, KB, 16), jnp.float32),
            pltpu.VMEM((3, KB, CW), jnp.float32),
            pltpu.VMEM((3, KB, CW), jnp.float32),
            pltpu.VMEM((3, KB), jnp.int32),
            pltpu.VMEM_SHARED((N_PAD, CW), jnp.float32),
            pltpu.VMEM_SHARED((N_PAD, 16), jnp.float32),
            [pltpu.SemaphoreType.DMA((3,))] * 6
            + [pltpu.SemaphoreType.DMA((8,))],
        ],
        compiler_params=pltpu.CompilerParams(use_tc_tiling_on_sc=False),
    )


def _fold_attn(w, a_src, a_dst, heads, out_ch):
    wr = w.reshape(w.shape[0], heads, out_ch)
    ws = (wr * a_src[None]).sum(-1)
    wd = (wr * a_dst[None]).sum(-1)
    return jnp.concatenate([w, ws, wd], axis=1)


def _expand_mat(heads, ch):
    e = jnp.zeros((16, heads * ch), jnp.float32)
    r = jnp.arange(heads * ch) // ch
    e = e.at[r, jnp.arange(heads * ch)].set(1.0)
    return e


def _split_m(m, heads, ch):
    hc = heads * ch
    nc = hc // CW
    h4 = m[:, :hc].reshape(N_NODES * nc, CW)
    atab = jnp.pad(m[:, hc:hc + heads], ((0, 0), (0, 16 - heads)))
    dtab = jnp.pad(m[:, hc + heads:hc + 2 * heads], ((0, 0), (0, 16 - heads)))
    return h4, atab, dtab


def kernel(x, params, edge_index):
    p = params
    n = N_NODES
    loop = jnp.arange(n, dtype=edge_index.dtype)
    padz = jnp.zeros((E_PAD - E_REAL,), edge_index.dtype)
    srcr = jnp.concatenate([edge_index[0], loop, padz]).reshape(16 * NB, KB)
    dstr = jnp.concatenate([edge_index[1], loop, padz]).reshape(16 * NB, KB)

    w1 = _fold_attn(p["c1_w"], p["c1_as"], p["c1_ad"], 8, 32)
    w2 = _fold_attn(p["c2_w"], p["c2_as"], p["c2_ad"], 8, 32)
    w3 = _fold_attn(p["c3_w"], p["c3_as"], p["c3_ad"], 12, 64)
    exp8 = _expand_mat(8, 32)
    exp12 = _expand_mat(12, 64)

    m1, gg2 = pl.pallas_call(
        _tc1_body,
        grid=(n // BN,),
        in_specs=[_rows(128), _rep(128, 272), _rep(128, 32), _vec(32),
                  _rep(32, 32), _vec(32)],
        out_specs=[_rows(272), _rows(32)],
        out_shape=[jax.ShapeDtypeStruct((n, 272), jnp.float32),
                   jax.ShapeDtypeStruct((n, 32), jnp.float32)],
    )(x, w1, p["ln1_w"], p["ln1_b"], p["ln2_w"], p["ln2_b"])

    gat1_sc = _make_gat_sc(8, 32)
    gat3_sc = _make_gat_sc(12, 64)

    zac = jnp.zeros((ROWS_T, CW), jnp.float32)
    zde = jnp.zeros((ROWS_T, 16), jnp.float32)

    h4, atab, dtab = _split_m(m1, 8, 32)
    acc1, den1, _w1 = gat1_sc(h4, atab, dtab, srcr, dstr, zac, zde)
    acc1, den1 = acc1[:, :n], den1[:n]

    m2, xa1 = pl.pallas_call(
        functools.partial(_tc_mid_body, nc=4),
        grid=(n // BN,),
        in_specs=[pl.BlockSpec((4, BN, CW), lambda i: (0, i, 0)),
                  _rows(16), _rep(16, 256), _vec(256),
                  _rep(256, 272), _rep(256, 80), _vec(80)],
        out_specs=[_rows(272), _rows(80)],
        out_shape=[jax.ShapeDtypeStruct((n, 272), jnp.float32),
                   jax.ShapeDtypeStruct((n, 80), jnp.float32)],
    )(acc1, den1, exp8, p["c1_b"], w2, p["a1_w"], p["a1_b"])

    h4, atab, dtab = _split_m(m2, 8, 32)
    acc2, den2, _w2 = gat1_sc(h4, atab, dtab, srcr, dstr, zac, zde)
    acc2, den2 = acc2[:, :n], den2[:n]

    m3, xa2 = pl.pallas_call(
        functools.partial(_tc_mid_body, nc=4),
        grid=(n // BN,),
        in_specs=[pl.BlockSpec((4, BN, CW), lambda i: (0, i, 0)),
                  _rows(16), _rep(16, 256), _vec(256),
                  _rep(256, 792), _rep(256, 200), _vec(200)],
        out_specs=[_rows(792), _rows(200)],
        out_shape=[jax.ShapeDtypeStruct((n, 792), jnp.float32),
                   jax.ShapeDtypeStruct((n, 200), jnp.float32)],
    )(acc2, den2, exp8, p["c2_b"], w3, p["a3_w"], p["a3_b"])

    h4, atab, dtab = _split_m(m3, 12, 64)
    acc3, den3, _w3 = gat3_sc(h4, atab, dtab, srcr, dstr, zac, zde)
    acc3, den3 = acc3[:, :n], den3[:n]

    f1 = p["f1_w"]
    out = pl.pallas_call(
        functools.partial(_tc_head_body, nc=12),
        grid=(n // BN,),
        in_specs=[pl.BlockSpec((12, BN, CW), lambda i: (0, i, 0)),
                  _rows(16), _rep(16, 768), _vec(768),
                  _rows(32), _rows(80), _rows(200),
                  _rep(768, 200), _rep(32, 200), _rep(80, 200),
                  _rep(200, 200), _vec(200),
                  _rep(200, 64), _vec(64), _rep(64, 1), _vec(1)],
        out_specs=_rows(1),
        out_shape=jax.ShapeDtypeStruct((n, 1), jnp.float32),
    )(acc3, den3, exp12, p["c3_b"], gg2, xa1, xa2,
      f1[32:800], f1[:32], f1[800:880], f1[880:1080], p["f1_b"],
      p["f2_w"], p["f2_b"], p["f3_w"], p["f3_b"])
    return out


# speedup vs baseline: 24.0391x; 1.0109x over previous
import functools

import jax
import jax.numpy as jnp
from jax import lax
from jax.experimental import pallas as pl
from jax.experimental.pallas import tpu as pltpu
from jax.experimental.pallas import tpu_sc as plsc

N_NODES = 10000
N_PAD = 10240
E_REAL = 330000
KB = 128
NB = 168
PT = NB * KB
E_PAD = 16 * PT
ROWS_T = N_PAD // 16
BN = 1000


def _mm_body(x_ref, w_ref, b_ref, o_ref, *, act):
    acc = jnp.dot(x_ref[...], w_ref[...], preferred_element_type=jnp.float32)
    acc = acc + b_ref[...][None, :]
    if act == "relu":
        acc = jnp.maximum(acc, 0.0)
    o_ref[...] = acc


def _mm(x, w, b, act="none"):
    n, k = x.shape
    m = w.shape[1]
    return pl.pallas_call(
        functools.partial(_mm_body, act=act),
        grid=(n // BN,),
        in_specs=[
            pl.BlockSpec((BN, k), lambda i: (i, 0)),
            pl.BlockSpec((k, m), lambda i: (0, 0)),
            pl.BlockSpec((m,), lambda i: (0,)),
        ],
        out_specs=pl.BlockSpec((BN, m), lambda i: (i, 0)),
        out_shape=jax.ShapeDtypeStruct((n, m), jnp.float32),
    )(x, w, b)


def _tc1_body(x_ref, w1_ref, ln1w_ref, ln1b_ref, ln2w_ref, ln2b_ref,
              m1_ref, gg2_ref):
    x = x_ref[...]
    m1_ref[...] = jnp.dot(x, w1_ref[...], preferred_element_type=jnp.float32)
    gg1 = jnp.maximum(jnp.dot(x, ln1w_ref[...],
                              preferred_element_type=jnp.float32)
                      + ln1b_ref[...][None, :], 0.0)
    gg2_ref[...] = jnp.maximum(jnp.dot(gg1, ln2w_ref[...],
                                       preferred_element_type=jnp.float32)
                               + ln2b_ref[...][None, :], 0.0)


def _tc_mid_body(acc_ref, den_ref, exp_ref, bias_ref, w_ref, wa_ref, ba_ref,
                 m_ref, xa_ref, *, nc):
    acc = jnp.concatenate([acc_ref[i] for i in range(nc)], axis=-1)
    dd = jnp.dot(den_ref[...], exp_ref[...],
                 preferred_element_type=jnp.float32)
    xl = jnp.maximum(acc / dd + bias_ref[...][None, :], 0.0)
    m_ref[...] = jnp.dot(xl, w_ref[...], preferred_element_type=jnp.float32)
    xa_ref[...] = jnp.maximum(jnp.dot(xl, wa_ref[...],
                                      preferred_element_type=jnp.float32)
                              + ba_ref[...][None, :], 0.0)


def _tc_head_body(acc_ref, den_ref, exp_ref, bias_ref, gg2_ref, xa1_ref,
                  xa2_ref, f1x_ref, f1g_ref, f1a1_ref, f1a2_ref, f1b_ref,
                  f2w_ref, f2b_ref, f3w_ref, f3b_ref, o_ref, *, nc):
    acc = jnp.concatenate([acc_ref[i] for i in range(nc)], axis=-1)
    dd = jnp.dot(den_ref[...], exp_ref[...],
                 preferred_element_type=jnp.float32)
    x3 = jnp.maximum(acc / dd + bias_ref[...][None, :], 0.0)
    xf = (jnp.dot(x3, f1x_ref[...], preferred_element_type=jnp.float32)
          + jnp.dot(gg2_ref[...], f1g_ref[...],
                    preferred_element_type=jnp.float32)
          + jnp.dot(xa1_ref[...], f1a1_ref[...],
                    preferred_element_type=jnp.float32)
          + jnp.dot(xa2_ref[...], f1a2_ref[...],
                    preferred_element_type=jnp.float32)
          + f1b_ref[...][None, :])
    xf = jnp.maximum(xf, 0.0)
    xf = jnp.maximum(jnp.dot(xf, f2w_ref[...],
                             preferred_element_type=jnp.float32)
                     + f2b_ref[...][None, :], 0.0)
    xf = jnp.dot(xf, f3w_ref[...], preferred_element_type=jnp.float32) \
        + f3b_ref[...][None, :]
    o_ref[...] = jax.nn.sigmoid(xf)


def _rep(k, m):
    return pl.BlockSpec((k, m), lambda i: (0, 0))


def _vec(m):
    return pl.BlockSpec((m,), lambda i: (0,))


def _rows(m):
    return pl.BlockSpec((BN, m), lambda i: (i, 0))


CW = 64
NV = CW // 16


@functools.lru_cache(maxsize=None)
def _make_gat_sc(heads, ch):
    shift = 5 if ch == 32 else 6
    nc = heads * ch // CW
    n_rounds = nc // 2
    n_al = CW // ch if ch < CW else 1
    mesh = plsc.VectorSubcoreMesh(core_axis_name="c", subcore_axis_name="s",
                                  num_cores=2, num_subcores=16)

    def body(h4, atab, dtab, srcr, dstr, zac, zde, acc_hbm, den_hbm, w_hbm,
             srcb, dstb, asb, adb, wb, hbuf, obuf, ridx,
             acc, den, sems):
        g = lax.axis_index("c")
        sid = lax.axis_index("s")
        rowblk = sid * NB

        nrow0 = sid * ROWS_T
        pltpu.sync_copy(zde, den.at[pl.ds(nrow0, ROWS_T)])

        ebase0 = sid * PT
        sem_a, sem_d, sem_h, sem_ws, sem_ww, sem_ha, sem_e = sems

        def drain(kind, sl):
            if kind == "a":
                pltpu.make_async_copy(atab.at[pl.ds(0, KB)], asb.at[sl],
                                      sem_a.at[sl]).wait()
            elif kind == "d":
                pltpu.make_async_copy(dtab.at[pl.ds(0, KB)], adb.at[sl],
                                      sem_d.at[sl]).wait()
            elif kind == "h":
                pltpu.make_async_copy(h4.at[pl.ds(0, KB)], hbuf.at[sl],
                                      sem_h.at[sl]).wait()
            elif kind == "ws":
                pltpu.make_async_copy(wb.at[sl], den.at[pl.ds(0, KB)],
                                      sem_ws.at[sl]).wait()
            elif kind == "ww":
                pltpu.make_async_copy(wb.at[sl], w_hbm.at[pl.ds(0, KB)],
                                      sem_ww.at[sl]).wait()
            elif kind == "ha":
                pltpu.make_async_copy(obuf.at[sl], acc.at[pl.ds(0, KB)],
                                      sem_ha.at[sl]).wait()
            elif kind == "e":
                pltpu.make_async_copy(srcr.at[0], srcb.at[sl],
                                      sem_e.at[sl]).wait()
                pltpu.make_async_copy(srcr.at[0], srcb.at[sl],
                                      sem_e.at[sl]).wait()

        def pf_edges(b):
            el = jnp.bitwise_and(b, 7)
            pltpu.async_copy(srcr.at[rowblk + b], srcb.at[el], sem_e.at[el])
            pltpu.async_copy(dstr.at[rowblk + b], dstb.at[el], sem_e.at[el])

        for r in range(n_rounds):
            c = g * n_rounds + r
            pltpu.sync_copy(zac, acc.at[pl.ds(nrow0, ROWS_T)])
            plsc.subcore_barrier()

            hgs = [jnp.full((16,), lax.shift_right_logical(
                c * CW + a * ch, shift), jnp.int32) for a in range(n_al)]

            def pf_gather(b, sl):
                el = jnp.bitwise_and(b, 7)
                drain("e", el)
                if r == 0:
                    pltpu.async_copy(atab.at[srcb.at[el]], asb.at[sl],
                                     sem_a.at[sl])
                    pltpu.async_copy(dtab.at[dstb.at[el]], adb.at[sl],
                                     sem_d.at[sl])
                else:
                    pltpu.async_copy(
                        w_hbm.at[pl.ds(ebase0 + b * KB, KB)], wb.at[sl],
                        sem_a.at[sl])

                @plsc.parallel_loop(0, KB, 16, unroll=2)
                def rix(i):
                    sv = srcb[el, pl.ds(i, 16)]
                    ridx[sl, pl.ds(i, 16)] = sv * nc + c
                pltpu.async_copy(h4.at[ridx.at[sl]], hbuf.at[sl],
                                 sem_h.at[sl])

            pf_edges(0)
            pf_edges(1)
            pf_edges(2)
            pf_gather(0, jnp.int32(0))
            pf_gather(1, jnp.int32(1))
            if r > 0:
                pf_edges(3)
                pf_gather(2, jnp.int32(2))

            def section(b, sl):
                el = jnp.bitwise_and(b, 7)
                not_first = b >= 3
                drain("a", sl)
                if r == 0:
                    drain("d", sl)
                drain("h", sl)

                @pl.when(not_first)
                def _drains():
                    if r == 0:
                        drain("ws", sl)
                        drain("ww", sl)
                    drain("ha", sl)

                if r == 0:
                    ebase = ebase0 + b * KB

                    @plsc.parallel_loop(0, KB, 1, unroll=4)
                    def edge(e):
                        ev = asb[sl, e] + adb[sl, e]
                        ev = jnp.where(ev >= 0.0, ev, 0.2 * ev)
                        scale = jnp.where(ebase + e < E_REAL, 1.0, 0.0)
                        wv = jnp.exp(ev) * scale
                        wb[sl, e] = wv
                        for v in range(NV):
                            al = wv.at[hgs[(v * 16) // ch]].get(
                                mode="promise_in_bounds")
                            obuf[sl, e, pl.ds(v * 16, 16)] = (
                                hbuf[sl, e, pl.ds(v * 16, 16)] * al)
                    pltpu.async_copy(wb.at[sl], den.at[dstb.at[el]],
                                     sem_ws.at[sl], add=True)
                    pltpu.async_copy(wb.at[sl],
                                     w_hbm.at[pl.ds(ebase0 + b * KB, KB)],
                                     sem_ww.at[sl])
                else:
                    @plsc.parallel_loop(0, KB, 1, unroll=8)
                    def edge(e):
                        wv = wb[sl, e]
                        for v in range(NV):
                            al = wv.at[hgs[(v * 16) // ch]].get(
                                mode="promise_in_bounds")
                            obuf[sl, e, pl.ds(v * 16, 16)] = (
                                hbuf[sl, e, pl.ds(v * 16, 16)] * al)

                pltpu.async_copy(obuf.at[sl], acc.at[dstb.at[el]],
                                 sem_ha.at[sl], add=True)

                pfd = 2 if r == 0 else 3

                @pl.when(b + pfd + 1 < NB)
                def _pfe():
                    pf_edges(b + pfd + 1)

                @pl.when(b + pfd < NB)
                def _pf():
                    pf_gather(b + pfd, lax.rem(b + pfd, jnp.int32(3)))

            def blk(b, carry):
                section(b, lax.rem(b, jnp.int32(3)))
                return carry

            lax.fori_loop(0, NB, blk, 0)

            for sl in range(3):
                if r == 0:
                    drain("ws", sl)
                    drain("ww", sl)
                drain("ha", sl)
            plsc.subcore_barrier()
            pltpu.sync_copy(acc.at[pl.ds(nrow0, ROWS_T)],
                            acc_hbm.at[c, pl.ds(nrow0, ROWS_T)])
            if r == 0:
                @pl.when(g == 0)
                def _dump_den():
                    pltpu.sync_copy(den.at[pl.ds(nrow0, ROWS_T)],
                                    den_hbm.at[pl.ds(nrow0, ROWS_T)])

    return pl.kernel(
        body,
        out_type=(
            jax.ShapeDtypeStruct((nc, N_PAD, CW), jnp.float32),
            jax.ShapeDtypeStruct((N_PAD, 16), jnp.float32),
            jax.ShapeDtypeStruct((E_PAD, 16), jnp.float32),
        ),
        mesh=mesh,
        scratch_types=[
            pltpu.VMEM((8, KB), jnp.int32),
            pltpu.VMEM((8, KB), jnp.int32),
            pltpu.VMEM((3, KB, 16), jnp.float32),
            pltpu.VMEM((3, KB, 16), jnp.float32),
            pltpu.VMEM((3, KB, 16), jnp.float32),
            pltpu.VMEM((3, KB, CW), jnp.float32),
            pltpu.VMEM((3, KB, CW), jnp.float32),
            pltpu.VMEM((3, KB), jnp.int32),
            pltpu.VMEM_SHARED((N_PAD, CW), jnp.float32),
            pltpu.VMEM_SHARED((N_PAD, 16), jnp.float32),
            [pltpu.SemaphoreType.DMA((3,))] * 6
            + [pltpu.SemaphoreType.DMA((8,))],
        ],
        compiler_params=pltpu.CompilerParams(use_tc_tiling_on_sc=False),
    )


def _fold_attn(w, a_src, a_dst, heads, out_ch):
    wr = w.reshape(w.shape[0], heads, out_ch)
    ws = (wr * a_src[None]).sum(-1)
    wd = (wr * a_dst[None]).sum(-1)
    return jnp.concatenate([w, ws, wd], axis=1)


def _expand_mat(heads, ch):
    e = jnp.zeros((16, heads * ch), jnp.float32)
    r = jnp.arange(heads * ch) // ch
    e = e.at[r, jnp.arange(heads * ch)].set(1.0)
    return e


def _split_m(m, heads, ch):
    hc = heads * ch
    nc = hc // CW
    h4 = m[:, :hc].reshape(N_NODES * nc, CW)
    atab = jnp.pad(m[:, hc:hc + heads], ((0, 0), (0, 16 - heads)))
    dtab = jnp.pad(m[:, hc + heads:hc + 2 * heads], ((0, 0), (0, 16 - heads)))
    return h4, atab, dtab


def kernel(x, params, edge_index):
    p = params
    n = N_NODES
    loop = jnp.arange(n, dtype=edge_index.dtype)
    padz = jnp.zeros((E_PAD - E_REAL,), edge_index.dtype)
    srcr = jnp.concatenate([edge_index[0], loop, padz]).reshape(16 * NB, KB)
    dstr = jnp.concatenate([edge_index[1], loop, padz]).reshape(16 * NB, KB)

    w1 = _fold_attn(p["c1_w"], p["c1_as"], p["c1_ad"], 8, 32)
    w2 = _fold_attn(p["c2_w"], p["c2_as"], p["c2_ad"], 8, 32)
    w3 = _fold_attn(p["c3_w"], p["c3_as"], p["c3_ad"], 12, 64)
    exp8 = _expand_mat(8, 32)
    exp12 = _expand_mat(12, 64)

    m1, gg2 = pl.pallas_call(
        _tc1_body,
        grid=(n // BN,),
        in_specs=[_rows(128), _rep(128, 272), _rep(128, 32), _vec(32),
                  _rep(32, 32), _vec(32)],
        out_specs=[_rows(272), _rows(32)],
        out_shape=[jax.ShapeDtypeStruct((n, 272), jnp.float32),
                   jax.ShapeDtypeStruct((n, 32), jnp.float32)],
    )(x, w1, p["ln1_w"], p["ln1_b"], p["ln2_w"], p["ln2_b"])

    gat1_sc = _make_gat_sc(8, 32)
    gat3_sc = _make_gat_sc(12, 64)

    zac = jnp.zeros((ROWS_T, CW), jnp.float32)
    zde = jnp.zeros((ROWS_T, 16), jnp.float32)

    h4, atab, dtab = _split_m(m1, 8, 32)
    acc1, den1, _w1 = gat1_sc(h4, atab, dtab, srcr, dstr, zac, zde)
    acc1, den1 = acc1[:, :n], den1[:n]

    m2, xa1 = pl.pallas_call(
        functools.partial(_tc_mid_body, nc=4),
        grid=(n // BN,),
        in_specs=[pl.BlockSpec((4, BN, CW), lambda i: (0, i, 0)),
                  _rows(16), _rep(16, 256), _vec(256),
                  _rep(256, 272), _rep(256, 80), _vec(80)],
        out_specs=[_rows(272), _rows(80)],
        out_shape=[jax.ShapeDtypeStruct((n, 272), jnp.float32),
                   jax.ShapeDtypeStruct((n, 80), jnp.float32)],
    )(acc1, den1, exp8, p["c1_b"], w2, p["a1_w"], p["a1_b"])

    h4, atab, dtab = _split_m(m2, 8, 32)
    acc2, den2, _w2 = gat1_sc(h4, atab, dtab, srcr, dstr, zac, zde)
    acc2, den2 = acc2[:, :n], den2[:n]

    m3, xa2 = pl.pallas_call(
        functools.partial(_tc_mid_body, nc=4),
        grid=(n // BN,),
        in_specs=[pl.BlockSpec((4, BN, CW), lambda i: (0, i, 0)),
                  _rows(16), _rep(16, 256), _vec(256),
                  _rep(256, 792), _rep(256, 200), _vec(200)],
        out_specs=[_rows(792), _rows(200)],
        out_shape=[jax.ShapeDtypeStruct((n, 792), jnp.float32),
                   jax.ShapeDtypeStruct((n, 200), jnp.float32)],
    )(acc2, den2, exp8, p["c2_b"], w3, p["a3_w"], p["a3_b"])

    h4, atab, dtab = _split_m(m3, 12, 64)
    acc3, den3, _w3 = gat3_sc(h4, atab, dtab, srcr, dstr, zac, zde)
    acc3, den3 = acc3[:, :n], den3[:n]

    f1 = p["f1_w"]
    out = pl.pallas_call(
        functools.partial(_tc_head_body, nc=12),
        grid=(n // BN,),
        in_specs=[pl.BlockSpec((12, BN, CW), lambda i: (0, i, 0)),
                  _rows(16), _rep(16, 768), _vec(768),
                  _rows(32), _rows(80), _rows(200),
                  _rep(768, 200), _rep(32, 200), _rep(80, 200),
                  _rep(200, 200), _vec(200),
                  _rep(200, 64), _vec(64), _rep(64, 1), _vec(1)],
        out_specs=_rows(1),
        out_shape=jax.ShapeDtypeStruct((n, 1), jnp.float32),
    )(acc3, den3, exp12, p["c3_b"], gg2, xa1, xa2,
      f1[32:800], f1[:32], f1[800:880], f1[880:1080], p["f1_b"],
      p["f2_w"], p["f2_b"], p["f3_w"], p["f3_b"])
    return out
